# Initial kernel scaffold; baseline (speedup 1.0000x reference)
import functools

import jax
import jax.numpy as jnp
from jax import lax
from jax.experimental import pallas as pl
from jax.experimental.pallas import tpu as pltpu
from jax.experimental.pallas import tpu_sc as plsc

NN = 10000
NP = 10240
HH = 5
NB = 64
EBLK = 128
NCC = 2
NSS = 16
NWK = NCC * NSS
RPT = NP // NSS
CH = 160

_F32 = jnp.float32


def _bcast(v16, r):
    return jnp.take(v16, jnp.full((16,), r, jnp.int32), mode="promise_in_bounds")


def _zero_ref(ref, nwords):
    def body(i, _):
        ref[pl.ds(i * 16, 16)] = jnp.zeros((16,), _F32)
        return 0
    lax.fori_loop(0, nwords // 16, body, 0)


def _zero_2d(ref, rows, cols):
    def body(i, _):
        for c in range(cols // 16):
            ref[i, pl.ds(c * 16, 16)] = jnp.zeros((16,), _F32)
        return 0
    lax.fori_loop(0, rows, body, 0)


def _add_into(dst, src, nwords):
    def body(i, _):
        sl = pl.ds(i * 16, 16)
        dst[sl] = dst[sl] + src[sl]
        return 0
    lax.fori_loop(0, nwords // 16, body, 0)


def _build_deg(etp):
    epw = etp // NWK
    nblk = epw // EBLK
    mesh = plsc.VectorSubcoreMesh(core_axis_name="c", subcore_axis_name="s")

    @functools.partial(
        pl.kernel, mesh=mesh,
        out_type=jax.ShapeDtypeStruct((NCC, NP), _F32),
        scratch_types=[
            pltpu.VMEM((NP,), _F32),
            pltpu.VMEM((EBLK,), jnp.int32),
            pltpu.VMEM((RPT,), _F32),
            pltpu.VMEM((RPT,), _F32),
            pltpu.VMEM_SHARED((NSS, NP), _F32),
        ],
    )
    def deg_kernel(colp_hbm, out_hbm, degv, cidx, redv, tmpv, part_sh):
        cid = lax.axis_index("c")
        sid = lax.axis_index("s")
        wid = cid * NSS + sid
        base = wid * epw
        _zero_ref(degv, NP)
        ones = jnp.ones((16,), _F32)

        def blk(b, _):
            pltpu.sync_copy(colp_hbm.at[pl.ds(base + b * EBLK, EBLK)], cidx)
            def grp(j, _):
                c16 = cidx[pl.ds(j * 16, 16)]
                plsc.addupdate_scatter(degv, [c16], ones)
                return 0
            lax.fori_loop(0, EBLK // 16, grp, 0)
            return 0
        lax.fori_loop(0, nblk, blk, 0)

        pltpu.sync_copy(degv, part_sh.at[sid])
        plsc.subcore_barrier()
        rbase = sid * RPT
        _zero_ref(redv, RPT)
        def radd(t, _):
            pltpu.sync_copy(part_sh.at[t, pl.ds(rbase, RPT)], tmpv)
            _add_into(redv, tmpv, RPT)
            return 0
        lax.fori_loop(0, NSS, radd, 0)
        pltpu.sync_copy(redv, out_hbm.at[cid, pl.ds(rbase, RPT)])

    return deg_kernel


def _build_den(etp):
    epw = etp // NWK
    nblk = epw // EBLK
    mesh = plsc.VectorSubcoreMesh(core_axis_name="c", subcore_axis_name="s")

    @functools.partial(
        pl.kernel, mesh=mesh,
        out_type=jax.ShapeDtypeStruct((NCC, HH, NP), _F32),
        scratch_types=[
            pltpu.VMEM((NN,), _F32),
            pltpu.VMEM((NP,), _F32),
            pltpu.VMEM((NP,), _F32),
            pltpu.VMEM((EBLK,), jnp.int32),
            pltpu.VMEM((EBLK,), jnp.int32),
            pltpu.VMEM((RPT,), _F32),
            pltpu.VMEM((RPT,), _F32),
            pltpu.VMEM_SHARED((NSS, NP), _F32),
        ],
    )
    def den_kernel(ast_hbm, adt_hbm, rowp_hbm, colp_hbm, out_hbm,
                   asv, adv, denv, ridx, cidx, redv, tmpv, part_sh):
        cid = lax.axis_index("c")
        sid = lax.axis_index("s")
        wid = cid * NSS + sid
        base = wid * epw
        rbase = sid * RPT

        for k in range(HH):
            pltpu.sync_copy(ast_hbm.at[k], asv)
            pltpu.sync_copy(adt_hbm.at[k], adv)
            _zero_ref(denv, NP)

            def blk(b, _):
                pltpu.sync_copy(rowp_hbm.at[pl.ds(base + b * EBLK, EBLK)], ridx)
                pltpu.sync_copy(colp_hbm.at[pl.ds(base + b * EBLK, EBLK)], cidx)
                def grp(j, _):
                    gsl = pl.ds(j * 16, 16)
                    r16 = ridx[gsl]
                    c16 = cidx[gsl]
                    a16 = plsc.load_gather(asv, [r16]) + plsc.load_gather(adv, [c16])
                    a16 = jnp.where(a16 > 0, a16, 0.2 * a16)
                    ex = jnp.exp(a16)
                    plsc.addupdate_scatter(denv, [c16], ex)
                    return 0
                lax.fori_loop(0, EBLK // 16, grp, 0)
                return 0
            lax.fori_loop(0, nblk, blk, 0)

            pltpu.sync_copy(denv, part_sh.at[sid])
            plsc.subcore_barrier()
            _zero_ref(redv, RPT)
            def radd(t, _):
                pltpu.sync_copy(part_sh.at[t, pl.ds(rbase, RPT)], tmpv)
                _add_into(redv, tmpv, RPT)
                return 0
            lax.fori_loop(0, NSS, radd, 0)
            pltpu.sync_copy(redv, out_hbm.at[cid, k, pl.ds(rbase, RPT)])
            plsc.subcore_barrier()

    return den_kernel


def _build_prop(etp, d, mode):
    epw = etp // NWK
    nblk = epw // EBLK
    mesh = plsc.VectorSubcoreMesh(core_axis_name="c", subcore_axis_name="s")

    if mode == "gcn":
        out_type = jax.ShapeDtypeStruct((NCC, NP, d), _F32)
    else:
        out_type = jax.ShapeDtypeStruct((HH, NCC, NP, d), _F32)

    scratch = [
        pltpu.VMEM((EBLK, d), _F32),
        pltpu.VMEM((CH, d), _F32),
        pltpu.VMEM((EBLK,), jnp.int32),
        pltpu.VMEM((EBLK,), jnp.int32),
        pltpu.VMEM((EBLK,), _F32),
        pltpu.VMEM_SHARED((NP, d), _F32),
        pltpu.SemaphoreType.DMA,
    ]
    if mode == "gcn":
        scratch = [pltpu.VMEM((NP,), _F32)] + scratch
    else:
        scratch = [pltpu.VMEM((NN,), _F32), pltpu.VMEM((NP,), _F32),
                   pltpu.VMEM((NP,), _F32), pltpu.VMEM((NP,), _F32)] + scratch

    def body(*refs):
        if mode == "gcn":
            (tbl_hbm, dis_hbm, rowp_hbm, colp_hbm, out_hbm,
             disv, rows_v, stg, ridx, cidx, wv, acc_sh, sem) = refs
        else:
            (tbl_hbm, ast_hbm, adt_hbm, denp_hbm, rowp_hbm, colp_hbm, out_hbm,
             asv, adv, denv, tmpnv, rows_v, stg, ridx, cidx, wv, acc_sh, sem) = refs

        cid = lax.axis_index("c")
        sid = lax.axis_index("s")
        wid = cid * NSS + sid
        base = wid * epw
        rbase = sid * RPT

        if mode == "gcn":
            pltpu.sync_copy(dis_hbm, disv)

        def one_pass(weight_fn, out_at):
            _zero_2d(stg, CH, d)
            for i in range(RPT // CH):
                pltpu.sync_copy(stg, acc_sh.at[pl.ds(rbase + i * CH, CH)])
            plsc.subcore_barrier()

            def blk(b, _):
                eb = base + b * EBLK
                pltpu.sync_copy(rowp_hbm.at[pl.ds(eb, EBLK)], ridx)
                pltpu.sync_copy(colp_hbm.at[pl.ds(eb, EBLK)], cidx)
                cp = pltpu.async_copy(tbl_hbm.at[ridx], rows_v, sem)
                def wgrp(j, _):
                    gsl = pl.ds(j * 16, 16)
                    wv[gsl] = weight_fn(ridx[gsl], cidx[gsl])
                    return 0
                lax.fori_loop(0, EBLK // 16, wgrp, 0)
                cp.wait()
                def mgrp(j, _):
                    w16 = wv[pl.ds(j * 16, 16)]
                    for r in range(16):
                        wb = _bcast(w16, r)
                        ri = j * 16 + r
                        for c in range(d // 16):
                            csl = pl.ds(c * 16, 16)
                            rows_v[ri, csl] = rows_v[ri, csl] * wb
                    return 0
                lax.fori_loop(0, EBLK // 16, mgrp, 0)
                pltpu.sync_copy(rows_v, acc_sh.at[cidx], add=True)
                return 0
            lax.fori_loop(0, nblk, blk, 0)
            plsc.subcore_barrier()
            for i in range(RPT // CH):
                dsl = pl.ds(rbase + i * CH, CH)
                pltpu.sync_copy(acc_sh.at[dsl], stg)
                pltpu.sync_copy(stg, out_at(dsl))
            plsc.subcore_barrier()

        if mode == "gcn":
            def wfn(r16, c16):
                return plsc.load_gather(disv, [r16]) * plsc.load_gather(disv, [c16])
            one_pass(wfn, lambda dsl: out_hbm.at[cid, dsl])
        else:
            for k in range(HH):
                pltpu.sync_copy(ast_hbm.at[k], asv)
                pltpu.sync_copy(adt_hbm.at[k], adv)
                pltpu.sync_copy(denp_hbm.at[0, k], denv)
                pltpu.sync_copy(denp_hbm.at[1, k], tmpnv)
                _add_into(denv, tmpnv, NP)
                def wfn(r16, c16):
                    a16 = (plsc.load_gather(asv, [r16]) +
                           plsc.load_gather(adv, [c16]))
                    a16 = jnp.where(a16 > 0, a16, 0.2 * a16)
                    ex = jnp.exp(a16)
                    den16 = plsc.load_gather(denv, [c16])
                    return ex / (den16 + 1e-16)
                one_pass(wfn, lambda dsl, k=k: out_hbm.at[k, cid, dsl])

    return functools.partial(pl.kernel, mesh=mesh, out_type=out_type,
                             scratch_types=scratch)(body)


def _gelu(t):
    return jax.nn.gelu(t, approximate=False)


def _bn_rows(h, g, b):
    mu = jnp.mean(h, axis=0)
    var = jnp.mean((h - mu) ** 2, axis=0)
    return (h - mu) / jnp.sqrt(var + 1e-5) * g + b


def _t_dis(degp):
    def body(degp_ref, dis_ref):
        deg = degp_ref[0, :] + degp_ref[1, :]
        dis_ref[...] = jnp.where(deg > 0, lax.rsqrt(deg), 0.0)
    return pl.pallas_call(
        body, out_shape=jax.ShapeDtypeStruct((NP,), _F32))(degp)


def _t1(p1, w1p, b1, g1, be1):
    def body(p_ref, w_ref, b_ref, g_ref, be_ref, h_ref):
        xx = p_ref[0, :NN, :] + p_ref[1, :NN, :]
        hh = jnp.dot(xx, w_ref[...], preferred_element_type=_F32) + b_ref[...]
        h_ref[...] = _gelu(_bn_rows(hh, g_ref[...], be_ref[...]))
    return pl.pallas_call(
        body, out_shape=jax.ShapeDtypeStruct((NN, 128), _F32))(
            p1, w1p, b1, g1, be1)


def _t2(p2, wh, bh, gh, beh, h1, wg3, ats, atd):
    def body(p_ref, w_ref, b_ref, g_ref, be_ref, h1_ref, wg_ref, ats_ref,
             atd_ref, h2_ref, ast_ref, adt_ref):
        xx = p_ref[0, :NN, :] + p_ref[1, :NN, :]
        hh = jnp.dot(xx, w_ref[...], preferred_element_type=_F32) + b_ref[...]
        h2 = _gelu(_bn_rows(hh, g_ref[...], be_ref[...])) + h1_ref[...]
        h2_ref[...] = h2
        va_s = jnp.sum(wg_ref[...] * ats_ref[...][None, :, :], axis=-1)
        va_d = jnp.sum(wg_ref[...] * atd_ref[...][None, :, :], axis=-1)
        a_sT = jax.lax.dot_general(va_s, h2, (((0,), (1,)), ((), ())),
                                   preferred_element_type=_F32)
        a_dT = jax.lax.dot_general(va_d, h2, (((0,), (1,)), ((), ())),
                                   preferred_element_type=_F32)
        ast_ref[...] = a_sT
        adt_ref[...] = jnp.concatenate(
            [a_dT, jnp.zeros((HH, NP - NN), _F32)], axis=1)
    return pl.pallas_call(
        body,
        out_shape=[jax.ShapeDtypeStruct((NN, 128), _F32),
                   jax.ShapeDtypeStruct((HH, NN), _F32),
                   jax.ShapeDtypeStruct((HH, NP), _F32)])(
            p2, wh, bh, gh, beh, h1, wg3, ats, atd)


def _t4(pg, wgh, bg):
    def body(p_ref, w_ref, bg_ref, g_ref):
        k = pl.program_id(0)
        pk = p_ref[0, 0, :NN, :] + p_ref[0, 1, :NN, :]
        contrib = jnp.dot(pk, w_ref[0], preferred_element_type=_F32) * (1.0 / HH)
        @pl.when(k == 0)
        def _():
            g_ref[...] = contrib + bg_ref[...]
        @pl.when(k > 0)
        def _():
            g_ref[...] = g_ref[...] + contrib
    return pl.pallas_call(
        body,
        grid=(HH,),
        in_specs=[pl.BlockSpec((1, NCC, NP, 128), lambda k: (k, 0, 0, 0)),
                  pl.BlockSpec((1, 128, 128), lambda k: (k, 0, 0)),
                  pl.BlockSpec((128,), lambda k: (0,))],
        out_specs=pl.BlockSpec((NN, 128), lambda k: (0, 0)),
        out_shape=jax.ShapeDtypeStruct((NN, 128), _F32))(pg, wgh, bg)


def _t5(p4, wo, bo, go, beo, batchf):
    def body(p_ref, w_ref, b_ref, g_ref, be_ref, bt_ref, s_ref, st_ref):
        xx = p_ref[0, :NN, :] + p_ref[1, :NN, :]
        hh = jnp.dot(xx, w_ref[...], preferred_element_type=_F32) + b_ref[...]
        hj = _gelu(_bn_rows(hh, g_ref[...], be_ref[...]))
        oh = (bt_ref[...] == lax.broadcasted_iota(_F32, (NN, NB), 1)
              ).astype(_F32)
        s_ref[...] = jax.lax.dot_general(oh, hj, (((0,), (0,)), ((), ())),
                                         preferred_element_type=_F32)
        stats = jnp.zeros((1, 128), _F32)
        stats = stats.at[0, 0].set(jnp.sum(hj))
        stats = stats.at[0, 1].set(jnp.sum(hj * hj))
        st_ref[...] = stats[None]
    return pl.pallas_call(
        body,
        grid=(4,),
        in_specs=[pl.BlockSpec((NCC, NP, 128), lambda j: (0, 0, 0)),
                  pl.BlockSpec((128, 128), lambda j: (0, j)),
                  pl.BlockSpec((128,), lambda j: (j,)),
                  pl.BlockSpec((128,), lambda j: (j,)),
                  pl.BlockSpec((128,), lambda j: (j,)),
                  pl.BlockSpec((NN, 1), lambda j: (0, 0))],
        out_specs=[pl.BlockSpec((NB, 128), lambda j: (0, j)),
                   pl.BlockSpec((1, 1, 128), lambda j: (j, 0, 0))],
        out_shape=[jax.ShapeDtypeStruct((NB, 512), _F32),
                   jax.ShapeDtypeStruct((4, 1, 128), _F32)])(
            p4, wo, bo, go, beo, batchf)


def _t6(s, stats, batchf, lnw, lnb, pw, pb, png, pnb, fw, fb, l2w, l2b, cw, cb):
    def body(s_ref, st_ref, bt_ref, lnw_ref, lnb_ref, pw_ref, pb_ref, png_ref,
             pnb_ref, fw_ref, fb_ref, l2w_ref, l2b_ref, cw_ref, cb_ref,
             xn_ref, c_ref):
        oh = (bt_ref[...] == lax.broadcasted_iota(_F32, (NN, NB), 1)
              ).astype(_F32)
        cnt = jnp.sum(oh, axis=0)[:, None]
        g = s_ref[...] / jnp.maximum(cnt, 1.0)
        tot = float(NN * 512)
        musum = jnp.sum(st_ref[:, 0, 0])
        sqsum = jnp.sum(st_ref[:, 0, 1])
        mu = musum / tot
        var = sqsum / tot - mu * mu
        g = (g - mu) / jnp.sqrt(var + 1e-5) * lnw_ref[...] + lnb_ref[...]
        p = jnp.dot(g, pw_ref[...], preferred_element_type=_F32) + pb_ref[...]
        p = _gelu(_bn_rows(p, png_ref[...], pnb_ref[...]))
        q = jnp.dot(p, fw_ref[...], preferred_element_type=_F32) + fb_ref[...] + p
        mu2 = jnp.mean(q, axis=-1, keepdims=True)
        var2 = jnp.mean((q - mu2) ** 2, axis=-1, keepdims=True)
        z = (q - mu2) / jnp.sqrt(var2 + 1e-5) * l2w_ref[...] + l2b_ref[...]
        nrm = jnp.sqrt(jnp.sum(z * z, axis=1, keepdims=True))
        xn = z / jnp.maximum(nrm, 1e-12)
        xn_ref[...] = xn
        lg = jnp.dot(xn, cw_ref[...], preferred_element_type=_F32) + cb_ref[...]
        m = jnp.max(lg, axis=1, keepdims=True)
        lse = m + jnp.log(jnp.sum(jnp.exp(lg - m), axis=1, keepdims=True))
        c_ref[...] = lg - lse
    return pl.pallas_call(
        body,
        out_shape=[jax.ShapeDtypeStruct((NB, 128), _F32),
                   jax.ShapeDtypeStruct((NB, 10), _F32)])(
            s, stats, batchf, lnw, lnb, pw, pb, png, pnb, fw, fb, l2w, l2b,
            cw, cb)


def kernel(x, W1, b1, g1, be1, Wh, bh, gh, beh, Wg, att_s, att_d, bg, Wo, bo,
           go, beo, lnw, lnb, pW, pb, png, pnb, fW, fb, l2w, l2b, cW, cb,
           edge_index, batch):
    n = x.shape[0]
    e = edge_index.shape[1]
    et = e + n
    ealign = NWK * EBLK
    etp = ((et + ealign - 1) // ealign) * ealign

    sl = jnp.arange(n, dtype=edge_index.dtype)
    rowp = jnp.pad(jnp.concatenate([edge_index[0], sl]), (0, etp - et))
    colp = jnp.pad(jnp.concatenate([edge_index[1], sl]), (0, etp - et),
                   constant_values=n)

    xpad = jnp.pad(x, ((0, 0), (0, 16 - x.shape[1])))
    w1p = jnp.pad(W1, ((0, 16 - W1.shape[0]), (0, 0)))
    wg3 = Wg.reshape(128, HH, 128)
    wgh = jnp.transpose(wg3, (1, 0, 2))
    batchf = batch.astype(_F32)[:, None]

    degp = _build_deg(etp)(colp)
    dis = _t_dis(degp)

    p1 = _build_prop(etp, 16, "gcn")(xpad, dis, rowp, colp)
    h1 = _t1(p1, w1p, b1, g1, be1)

    p2 = _build_prop(etp, 128, "gcn")(h1, dis, rowp, colp)
    h2, a_sT, a_dT = _t2(p2, Wh, bh, gh, beh, h1, wg3, att_s, att_d)

    denp = _build_den(etp)(a_sT, a_dT, rowp, colp)
    pg = _build_prop(etp, 128, "gat")(h2, a_sT, a_dT, denp, rowp, colp)
    g = _t4(pg, wgh, bg)

    p4 = _build_prop(etp, 128, "gcn")(g, dis, rowp, colp)
    s, stats = _t5(p4, Wo, bo, go, beo, batchf)
    xn, c = _t6(s, stats, batchf, lnw, lnb, pW, pb, png, pnb, fW, fb,
                l2w, l2b, cW, cb)
    return xn, c


# speedup vs baseline: 10.6035x; 10.6035x over previous
import functools

import jax
import jax.numpy as jnp
from jax import lax
from jax.experimental import pallas as pl
from jax.experimental.pallas import tpu as pltpu
from jax.experimental.pallas import tpu_sc as plsc

NN = 10000
NP = 10240
HH = 5
NB = 64
EBLK = 128
NCC = 2
NSS = 16
NWK = NCC * NSS
RPT = NP // NSS
CH = 160

_F32 = jnp.float32
_HI = lax.Precision.HIGHEST


def _rnd(a):
    return a.astype(jnp.bfloat16).astype(_F32)


_GDN = lax.GatherDimensionNumbers(offset_dims=(), collapsed_slice_dims=(0,),
                                  start_index_map=(0,))


def _bcast(v16, r):
    idx = jnp.full((16, 1), r, jnp.int32)
    return lax.gather(v16, idx, _GDN, (1,),
                      mode=lax.GatherScatterMode.PROMISE_IN_BOUNDS)


def _zero_ref(ref, nwords):
    def body(i, _):
        ref[pl.ds(i * 16, 16)] = jnp.zeros((16,), _F32)
        return 0
    lax.fori_loop(0, nwords // 16, body, 0)


def _zero_2d(ref, rows, cols):
    def body(i, _):
        for c in range(cols // 16):
            ref[i, pl.ds(c * 16, 16)] = jnp.zeros((16,), _F32)
        return 0
    lax.fori_loop(0, rows, body, 0)


def _add_into(dst, src, nwords):
    def body(i, _):
        sl = pl.ds(i * 16, 16)
        dst[sl] = dst[sl] + src[sl]
        return 0
    lax.fori_loop(0, nwords // 16, body, 0)


def _add_const(ref, nwords, val):
    def body(i, _):
        sl = pl.ds(i * 16, 16)
        ref[sl] = ref[sl] + val
        return 0
    lax.fori_loop(0, nwords // 16, body, 0)


def _build_deg(etp):
    epw = etp // NWK
    nblk = epw // EBLK
    mesh = plsc.VectorSubcoreMesh(core_axis_name="c", subcore_axis_name="s")

    @functools.partial(
        pl.kernel, mesh=mesh,
        compiler_params=pltpu.CompilerParams(needs_layout_passes=False),
        out_type=jax.ShapeDtypeStruct((NCC * NP,), _F32),
        scratch_types=[
            pltpu.VMEM((EBLK,), jnp.int32),
            pltpu.VMEM((EBLK,), _F32),
            pltpu.VMEM((RPT,), _F32),
            pltpu.VMEM_SHARED((NP,), _F32),
        ],
    )
    def deg_kernel(colp_hbm, out_hbm, cidx, onesv, stg, acc_sh):
        cid = lax.axis_index("c")
        sid = lax.axis_index("s")
        base = (cid * NSS + sid) * epw
        rbase = sid * RPT
        _zero_ref(stg, RPT)
        pltpu.sync_copy(stg, acc_sh.at[pl.ds(rbase, RPT)])
        _zero_ref(onesv, EBLK)
        _add_const(onesv, EBLK, 1.0)
        plsc.subcore_barrier()

        def blk(b, _):
            pltpu.sync_copy(colp_hbm.at[pl.ds(base + b * EBLK, EBLK)], cidx)
            pltpu.sync_copy(onesv, acc_sh.at[cidx], add=True)
            return 0
        lax.fori_loop(0, nblk, blk, 0)

        plsc.subcore_barrier()
        pltpu.sync_copy(acc_sh.at[pl.ds(rbase, RPT)], stg)
        pltpu.sync_copy(stg, out_hbm.at[pl.ds(cid * NP + rbase, RPT)])

    return deg_kernel


def _build_den(etp):
    epw = etp // NWK
    nblk = epw // EBLK
    mesh = plsc.VectorSubcoreMesh(core_axis_name="c", subcore_axis_name="s")

    @functools.partial(
        pl.kernel, mesh=mesh,
        compiler_params=pltpu.CompilerParams(needs_layout_passes=False),
        out_type=jax.ShapeDtypeStruct((NCC * HH * NP,), _F32),
        scratch_types=[
            pltpu.VMEM((NP,), _F32),
            pltpu.VMEM((NP,), _F32),
            pltpu.VMEM((EBLK,), jnp.int32),
            pltpu.VMEM((EBLK,), jnp.int32),
            pltpu.VMEM((EBLK,), _F32),
            pltpu.VMEM((RPT,), _F32),
            pltpu.VMEM_SHARED((NP,), _F32),
        ],
    )
    def den_kernel(ast_hbm, adt_hbm, rowp_hbm, colp_hbm, out_hbm,
                   asv, adv, ridx, cidx, exv, stg, acc_sh):
        cid = lax.axis_index("c")
        sid = lax.axis_index("s")
        base = (cid * NSS + sid) * epw
        rbase = sid * RPT

        for k in range(HH):
            pltpu.sync_copy(ast_hbm.at[pl.ds(k * NP, NP)], asv)
            pltpu.sync_copy(adt_hbm.at[pl.ds(k * NP, NP)], adv)
            _zero_ref(stg, RPT)
            pltpu.sync_copy(stg, acc_sh.at[pl.ds(rbase, RPT)])
            plsc.subcore_barrier()

            def blk(b, _):
                pltpu.sync_copy(rowp_hbm.at[pl.ds(base + b * EBLK, EBLK)], ridx)
                pltpu.sync_copy(colp_hbm.at[pl.ds(base + b * EBLK, EBLK)], cidx)
                def grp(j, _):
                    gsl = pl.ds(j * 16, 16)
                    a16 = (plsc.load_gather(asv, [ridx[gsl]]) +
                           plsc.load_gather(adv, [cidx[gsl]]))
                    a16 = jnp.where(a16 > 0, a16, 0.2 * a16)
                    exv[gsl] = jnp.exp(a16)
                    return 0
                lax.fori_loop(0, EBLK // 16, grp, 0)
                pltpu.sync_copy(exv, acc_sh.at[cidx], add=True)
                return 0
            lax.fori_loop(0, nblk, blk, 0)

            plsc.subcore_barrier()
            pltpu.sync_copy(acc_sh.at[pl.ds(rbase, RPT)], stg)
            pltpu.sync_copy(
                stg, out_hbm.at[pl.ds((cid * HH + k) * NP + rbase, RPT)])
            plsc.subcore_barrier()

    return den_kernel


def _build_wts_gcn(etp):
    epw = etp // NWK
    nblk = epw // EBLK
    mesh = plsc.VectorSubcoreMesh(core_axis_name="c", subcore_axis_name="s")

    @functools.partial(
        pl.kernel, mesh=mesh,
        compiler_params=pltpu.CompilerParams(needs_layout_passes=False),
        out_type=jax.ShapeDtypeStruct((etp,), _F32),
        scratch_types=[
            pltpu.VMEM((NP,), _F32),
            pltpu.VMEM((EBLK,), jnp.int32),
            pltpu.VMEM((EBLK,), jnp.int32),
            pltpu.VMEM((EBLK,), _F32),
        ],
    )
    def wts_kernel(dis_hbm, rowp_hbm, colp_hbm, out_hbm, disv, ridx, cidx, wb):
        cid = lax.axis_index("c")
        sid = lax.axis_index("s")
        base = (cid * NSS + sid) * epw
        pltpu.sync_copy(dis_hbm, disv)

        def blk(b, _):
            eb = base + b * EBLK
            pltpu.sync_copy(rowp_hbm.at[pl.ds(eb, EBLK)], ridx)
            pltpu.sync_copy(colp_hbm.at[pl.ds(eb, EBLK)], cidx)
            def grp(j, _):
                gsl = pl.ds(j * 16, 16)
                wb[gsl] = (plsc.load_gather(disv, [ridx[gsl]]) *
                           plsc.load_gather(disv, [cidx[gsl]]))
                return 0
            lax.fori_loop(0, EBLK // 16, grp, 0)
            pltpu.sync_copy(wb, out_hbm.at[pl.ds(eb, EBLK)])
            return 0
        lax.fori_loop(0, nblk, blk, 0)

    return wts_kernel


def _build_wts_gat(etp):
    epw = etp // NWK
    nblk = epw // EBLK
    mesh = plsc.VectorSubcoreMesh(core_axis_name="c", subcore_axis_name="s")

    @functools.partial(
        pl.kernel, mesh=mesh,
        compiler_params=pltpu.CompilerParams(needs_layout_passes=False),
        out_type=jax.ShapeDtypeStruct((HH * etp,), _F32),
        scratch_types=[
            pltpu.VMEM((NP,), _F32),
            pltpu.VMEM((NP,), _F32),
            pltpu.VMEM((NP,), _F32),
            pltpu.VMEM((EBLK,), jnp.int32),
            pltpu.VMEM((EBLK,), jnp.int32),
            pltpu.VMEM((EBLK,), _F32),
        ],
    )
    def wts_kernel(ast_hbm, adt_hbm, den_hbm, rowp_hbm, colp_hbm, out_hbm,
                   asv, adv, denv, ridx, cidx, wb):
        cid = lax.axis_index("c")
        sid = lax.axis_index("s")
        base = (cid * NSS + sid) * epw

        for k in range(HH):
            pltpu.sync_copy(ast_hbm.at[pl.ds(k * NP, NP)], asv)
            pltpu.sync_copy(adt_hbm.at[pl.ds(k * NP, NP)], adv)
            pltpu.sync_copy(den_hbm.at[pl.ds(k * NP, NP)], denv)

            def blk(b, _):
                eb = base + b * EBLK
                pltpu.sync_copy(rowp_hbm.at[pl.ds(eb, EBLK)], ridx)
                pltpu.sync_copy(colp_hbm.at[pl.ds(eb, EBLK)], cidx)
                def grp(j, _):
                    gsl = pl.ds(j * 16, 16)
                    r16 = ridx[gsl]
                    c16 = cidx[gsl]
                    a16 = (plsc.load_gather(asv, [r16]) +
                           plsc.load_gather(adv, [c16]))
                    a16 = jnp.where(a16 > 0, a16, 0.2 * a16)
                    ex = jnp.exp(a16)
                    den16 = plsc.load_gather(denv, [c16])
                    wb[gsl] = ex / (den16 + 1e-16)
                    return 0
                lax.fori_loop(0, EBLK // 16, grp, 0)
                pltpu.sync_copy(wb, out_hbm.at[pl.ds(k * etp + eb, EBLK)])
                return 0
            lax.fori_loop(0, nblk, blk, 0)

    return wts_kernel


def _build_prop(etp, mode):
    d = 128
    epw = etp // NWK
    nblk = epw // EBLK
    mesh = plsc.VectorSubcoreMesh(core_axis_name="c", subcore_axis_name="s")

    if mode == "gcn":
        out_type = jax.ShapeDtypeStruct((NCC, NP, d), _F32)
        wshape = (etp,)
    else:
        out_type = jax.ShapeDtypeStruct((HH, NCC, NP, d), _F32)
        wshape = (HH * etp,)

    @functools.partial(
        pl.kernel, mesh=mesh,
        compiler_params=pltpu.CompilerParams(needs_layout_passes=False),
        out_type=out_type,
        scratch_types=[
            pltpu.VMEM((EBLK, d), _F32),
            pltpu.VMEM((CH, d), _F32),
            pltpu.VMEM((EBLK,), jnp.int32),
            pltpu.VMEM((EBLK,), jnp.int32),
            pltpu.VMEM((EBLK,), _F32),
            pltpu.VMEM_SHARED((NP, d), _F32),
            pltpu.SemaphoreType.DMA,
        ],
    )
    def prop_kernel(tbl_hbm, w_hbm, rowp_hbm, colp_hbm, out_hbm,
                    rows_v, stg, ridx, cidx, wv, acc_sh, sem):
        cid = lax.axis_index("c")
        sid = lax.axis_index("s")
        base = (cid * NSS + sid) * epw
        rbase = sid * RPT

        def one_pass(woff, out_at):
            _zero_2d(stg, CH, d)
            for i in range(RPT // CH):
                pltpu.sync_copy(stg, acc_sh.at[pl.ds(rbase + i * CH, CH)])
            plsc.subcore_barrier()

            def blk(b, _):
                eb = base + b * EBLK
                pltpu.sync_copy(rowp_hbm.at[pl.ds(eb, EBLK)], ridx)
                pltpu.sync_copy(colp_hbm.at[pl.ds(eb, EBLK)], cidx)
                pltpu.sync_copy(w_hbm.at[pl.ds(woff + eb, EBLK)], wv)
                cp = pltpu.async_copy(tbl_hbm.at[ridx], rows_v, sem)
                cp.wait()
                def mgrp(j, _):
                    w16 = wv[pl.ds(j * 16, 16)]
                    for r in range(16):
                        wb = _bcast(w16, r)
                        ri = j * 16 + r
                        for c in range(d // 16):
                            csl = pl.ds(c * 16, 16)
                            rows_v[ri, csl] = rows_v[ri, csl] * wb
                    return 0
                lax.fori_loop(0, EBLK // 16, mgrp, 0)
                pltpu.sync_copy(rows_v, acc_sh.at[cidx], add=True)
                return 0
            lax.fori_loop(0, nblk, blk, 0)
            plsc.subcore_barrier()
            for i in range(RPT // CH):
                dsl = pl.ds(rbase + i * CH, CH)
                pltpu.sync_copy(acc_sh.at[dsl], stg)
                pltpu.sync_copy(stg, out_at(dsl))
            plsc.subcore_barrier()

        if mode == "gcn":
            one_pass(0, lambda dsl: out_hbm.at[cid, dsl])
        else:
            for k in range(HH):
                one_pass(k * etp, lambda dsl, k=k: out_hbm.at[k, cid, dsl])

    return prop_kernel


def _gelu(t):
    return 0.5 * t * (1.0 + lax.erf(t * (2.0 ** -0.5)))


def _bn_rows(h, g, b):
    mu = jnp.mean(h, axis=0)
    var = jnp.mean((h - mu) ** 2, axis=0)
    return (h - mu) / jnp.sqrt(var + 1e-5) * g + b


def _t_dis(degp):
    def body(degp_ref, dis_ref):
        deg = degp_ref[0, :] + degp_ref[1, :]
        dis_ref[...] = jnp.where(deg > 0, lax.rsqrt(deg), 0.0)
    return pl.pallas_call(
        body, out_shape=jax.ShapeDtypeStruct((NP,), _F32))(degp)


def _t0(x, w1):
    def body(x_ref, w_ref, o_ref):
        o_ref[...] = jnp.dot(x_ref[...], w_ref[...],
                             preferred_element_type=_F32)
    return pl.pallas_call(
        body, out_shape=jax.ShapeDtypeStruct((NN, 128), _F32))(x, w1)


def _t_den(denp):
    def body(dp_ref, d_ref):
        d_ref[...] = dp_ref[0, :] + dp_ref[1, :]
    return pl.pallas_call(
        body, out_shape=jax.ShapeDtypeStruct((HH * NP,), _F32))(denp)


def _t1(p1, b1, g1, be1):
    def body(p_ref, b_ref, g_ref, be_ref, h_ref):
        hh = p_ref[0, :NN, :] + p_ref[1, :NN, :] + b_ref[...]
        h_ref[...] = _gelu(_bn_rows(hh, g_ref[...], be_ref[...]))
    return pl.pallas_call(
        body, out_shape=jax.ShapeDtypeStruct((NN, 128), _F32))(
            p1, b1, g1, be1)


def _t2(p2, wh, bh, gh, beh, h1, wg3, ats, atd):
    def body(p_ref, w_ref, b_ref, g_ref, be_ref, h1_ref, wg_ref, ats_ref,
             atd_ref, h2_ref, ast_ref, adt_ref):
        xx = p_ref[0, :NN, :] + p_ref[1, :NN, :]
        hh = jnp.dot(xx, w_ref[...], preferred_element_type=_F32,
                     precision=_HI) + b_ref[...]
        h2 = _gelu(_bn_rows(hh, g_ref[...], be_ref[...])) + h1_ref[...]
        h2_ref[...] = h2
        h2r = h2.astype(jnp.bfloat16).astype(_F32)
        va_s = jnp.sum(wg_ref[...] * ats_ref[...][None, :, :], axis=-1)
        va_d = jnp.sum(wg_ref[...] * atd_ref[...][None, :, :], axis=-1)
        a_sT = jax.lax.dot_general(va_s, h2r, (((0,), (1,)), ((), ())),
                                   preferred_element_type=_F32,
                                   precision=_HI)
        a_dT = jax.lax.dot_general(va_d, h2r, (((0,), (1,)), ((), ())),
                                   preferred_element_type=_F32,
                                   precision=_HI)
        pad = jnp.zeros((HH, NP - NN), _F32)
        ast_ref[...] = jnp.concatenate([a_sT, pad], axis=1)
        adt_ref[...] = jnp.concatenate([a_dT, pad], axis=1)
    return pl.pallas_call(
        body,
        out_shape=[jax.ShapeDtypeStruct((NN, 128), _F32),
                   jax.ShapeDtypeStruct((HH, NP), _F32),
                   jax.ShapeDtypeStruct((HH, NP), _F32)])(
            p2, wh, bh, gh, beh, h1, wg3, ats, atd)


def _t4(pg, wgh, bg):
    def body(p_ref, w_ref, bg_ref, g_ref):
        k = pl.program_id(0)
        pk = p_ref[0, 0, :NN, :] + p_ref[0, 1, :NN, :]
        contrib = jnp.dot(pk, w_ref[0], preferred_element_type=_F32,
                          precision=_HI) * (1.0 / HH)
        @pl.when(k == 0)
        def _():
            g_ref[...] = contrib + bg_ref[...]
        @pl.when(k > 0)
        def _():
            g_ref[...] = g_ref[...] + contrib
    return pl.pallas_call(
        body,
        grid=(HH,),
        in_specs=[pl.BlockSpec((1, NCC, NP, 128), lambda k: (k, 0, 0, 0)),
                  pl.BlockSpec((1, 128, 128), lambda k: (k, 0, 0)),
                  pl.BlockSpec((128,), lambda k: (0,))],
        out_specs=pl.BlockSpec((NN, 128), lambda k: (0, 0)),
        out_shape=jax.ShapeDtypeStruct((NN, 128), _F32))(pg, wgh, bg)


def _t5(p4, wo, bo, go, beo, batchf):
    def body(p_ref, w_ref, b_ref, g_ref, be_ref, bt_ref, s_ref, st_ref):
        xx = p_ref[0, :NN, :] + p_ref[1, :NN, :]
        hh = jnp.dot(xx, w_ref[...], preferred_element_type=_F32,
                     precision=_HI) + b_ref[...]
        hj = _gelu(_bn_rows(hh, g_ref[...], be_ref[...]))
        oh = (bt_ref[...] == lax.broadcasted_iota(jnp.int32, (NN, NB), 1)
              .astype(_F32)).astype(_F32)
        s_ref[...] = jax.lax.dot_general(oh, hj, (((0,), (0,)), ((), ())),
                                         preferred_element_type=_F32,
                                         precision=_HI)
        lane = lax.broadcasted_iota(jnp.int32, (1, 1, 128), 2)
        stats = jnp.where(lane == 0, jnp.sum(hj),
                          jnp.where(lane == 1, jnp.sum(hj * hj), 0.0))
        st_ref[...] = stats
    return pl.pallas_call(
        body,
        grid=(4,),
        in_specs=[pl.BlockSpec((NCC, NP, 128), lambda j: (0, 0, 0)),
                  pl.BlockSpec((128, 128), lambda j: (0, j)),
                  pl.BlockSpec((128,), lambda j: (j,)),
                  pl.BlockSpec((128,), lambda j: (j,)),
                  pl.BlockSpec((128,), lambda j: (j,)),
                  pl.BlockSpec((NN, 1), lambda j: (0, 0))],
        out_specs=[pl.BlockSpec((NB, 128), lambda j: (0, j)),
                   pl.BlockSpec((1, 1, 128), lambda j: (j, 0, 0))],
        out_shape=[jax.ShapeDtypeStruct((NB, 512), _F32),
                   jax.ShapeDtypeStruct((4, 1, 128), _F32)])(
            p4, wo, bo, go, beo, batchf)


def _t6(s, stats, batchf, lnw, lnb, pw, pb, png, pnb, fw, fb, l2w, l2b, cw, cb):
    def body(s_ref, st_ref, bt_ref, lnw_ref, lnb_ref, pw_ref, pb_ref, png_ref,
             pnb_ref, fw_ref, fb_ref, l2w_ref, l2b_ref, cw_ref, cb_ref,
             xn_ref, c_ref):
        oh = (bt_ref[...] == lax.broadcasted_iota(jnp.int32, (NN, NB), 1)
              .astype(_F32)).astype(_F32)
        cnt = jnp.sum(oh, axis=0)[:, None]
        g = s_ref[...] / jnp.maximum(cnt, 1.0)
        tot = float(NN * 512)
        musum = jnp.sum(st_ref[:, 0, 0])
        sqsum = jnp.sum(st_ref[:, 0, 1])
        mu = musum / tot
        var = sqsum / tot - mu * mu
        g = (g - mu) / jnp.sqrt(var + 1e-5) * lnw_ref[...] + lnb_ref[...]
        p = jnp.dot(g, pw_ref[...], preferred_element_type=_F32) + pb_ref[...]
        p = _gelu(_bn_rows(p, png_ref[...], pnb_ref[...]))
        q = jnp.dot(p, fw_ref[...], preferred_element_type=_F32) + fb_ref[...] + p
        mu2 = jnp.mean(q, axis=-1, keepdims=True)
        var2 = jnp.mean((q - mu2) ** 2, axis=-1, keepdims=True)
        z = (q - mu2) / jnp.sqrt(var2 + 1e-5) * l2w_ref[...] + l2b_ref[...]
        nrm = jnp.sqrt(jnp.sum(z * z, axis=1, keepdims=True))
        xn = z / jnp.maximum(nrm, 1e-12)
        xn_ref[...] = xn
        lg = jnp.dot(xn, cw_ref[...], preferred_element_type=_F32) + cb_ref[...]
        m = jnp.max(lg, axis=1, keepdims=True)
        lse = m + jnp.log(jnp.sum(jnp.exp(lg - m), axis=1, keepdims=True))
        c_ref[...] = lg - lse
    return pl.pallas_call(
        body,
        out_shape=[jax.ShapeDtypeStruct((NB, 128), _F32),
                   jax.ShapeDtypeStruct((NB, 10), _F32)])(
            s, stats, batchf, lnw, lnb, pw, pb, png, pnb, fw, fb, l2w, l2b,
            cw, cb)


def kernel(x, W1, b1, g1, be1, Wh, bh, gh, beh, Wg, att_s, att_d, bg, Wo, bo,
           go, beo, lnw, lnb, pW, pb, png, pnb, fW, fb, l2w, l2b, cW, cb,
           edge_index, batch):
    n = x.shape[0]
    e = edge_index.shape[1]
    et = e + n
    ealign = NWK * EBLK
    etp = ((et + ealign - 1) // ealign) * ealign

    sl = jnp.arange(n, dtype=edge_index.dtype)
    rowp = jnp.pad(jnp.concatenate([edge_index[0], sl]), (0, etp - et))
    colp = jnp.pad(jnp.concatenate([edge_index[1], sl]), (0, etp - et),
                   constant_values=n)

    wg3_r = _rnd(Wg.reshape(128, HH, 128))
    wgh_r = jnp.transpose(wg3_r, (1, 0, 2))
    wh_r = _rnd(Wh)
    wo_r = _rnd(Wo)
    batchf = batch.astype(_F32)[:, None]

    degp = _build_deg(etp)(colp).reshape(NCC, NP)
    dis = _t_dis(degp)

    wgcn = _build_wts_gcn(etp)(dis, rowp, colp)

    x1 = _t0(x, W1)
    p1 = _build_prop(etp, "gcn")(x1, wgcn, rowp, colp)
    h1 = _t1(p1, b1, g1, be1)

    h1r = _rnd(h1)
    p2 = _build_prop(etp, "gcn")(h1r, wgcn, rowp, colp)
    h2, a_sT, a_dT = _t2(p2, wh_r, bh, gh, beh, h1, wg3_r, att_s, att_d)

    astf = a_sT.reshape(HH * NP)
    adtf = a_dT.reshape(HH * NP)
    denp = _build_den(etp)(astf, adtf, rowp, colp)
    den = _t_den(denp.reshape(NCC, HH * NP))
    wgat = _build_wts_gat(etp)(astf, adtf, den, rowp, colp)
    h2r = _rnd(h2)
    pg = _build_prop(etp, "gat")(h2r, wgat, rowp, colp)
    g = _t4(pg, wgh_r, bg)

    gr = _rnd(g)
    p4 = _build_prop(etp, "gcn")(gr, wgcn, rowp, colp)
    s, stats = _t5(p4, wo_r, bo, go, beo, batchf)
    xn, c = _t6(s, stats, batchf, lnw, lnb, pW, pb, png, pnb, fW, fb,
                l2w, l2b, cW, cb)
    return xn, c


# speedup vs baseline: 13.2775x; 1.2522x over previous
import functools

import jax
import jax.numpy as jnp
from jax import lax
from jax.experimental import pallas as pl
from jax.experimental.pallas import tpu as pltpu
from jax.experimental.pallas import tpu_sc as plsc

NN = 10000
NP = 10240
HH = 5
NB = 64
EBLK = 128
NCC = 2
NSS = 16
NWK = NCC * NSS
RPT = NP // NSS
CH = 80

_F32 = jnp.float32
_HI = lax.Precision.HIGHEST


def _rnd(a):
    return a.astype(jnp.bfloat16).astype(_F32)


_GDN = lax.GatherDimensionNumbers(offset_dims=(), collapsed_slice_dims=(0,),
                                  start_index_map=(0,))


def _bcast(v16, r):
    idx = jnp.full((16, 1), r, jnp.int32)
    return lax.gather(v16, idx, _GDN, (1,),
                      mode=lax.GatherScatterMode.PROMISE_IN_BOUNDS)


def _zero_ref(ref, nwords):
    def body(i, _):
        ref[pl.ds(i * 16, 16)] = jnp.zeros((16,), _F32)
        return 0
    lax.fori_loop(0, nwords // 16, body, 0)


def _zero_2d(ref, rows, cols):
    def body(i, _):
        for c in range(cols // 16):
            ref[i, pl.ds(c * 16, 16)] = jnp.zeros((16,), _F32)
        return 0
    lax.fori_loop(0, rows, body, 0)


def _add_into(dst, src, nwords):
    def body(i, _):
        sl = pl.ds(i * 16, 16)
        dst[sl] = dst[sl] + src[sl]
        return 0
    lax.fori_loop(0, nwords // 16, body, 0)


def _add_const(ref, nwords, val):
    def body(i, _):
        sl = pl.ds(i * 16, 16)
        ref[sl] = ref[sl] + val
        return 0
    lax.fori_loop(0, nwords // 16, body, 0)


def _build_deg(etp):
    epw = etp // NWK
    nblk = epw // EBLK
    mesh = plsc.VectorSubcoreMesh(core_axis_name="c", subcore_axis_name="s")

    @functools.partial(
        pl.kernel, mesh=mesh,
        compiler_params=pltpu.CompilerParams(needs_layout_passes=False),
        out_type=jax.ShapeDtypeStruct((NCC * NP,), _F32),
        scratch_types=[
            pltpu.VMEM((EBLK,), jnp.int32),
            pltpu.VMEM((EBLK,), _F32),
            pltpu.VMEM((RPT,), _F32),
            pltpu.VMEM_SHARED((NP,), _F32),
        ],
    )
    def deg_kernel(colp_hbm, out_hbm, cidx, onesv, stg, acc_sh):
        cid = lax.axis_index("c")
        sid = lax.axis_index("s")
        base = (cid * NSS + sid) * epw
        rbase = sid * RPT
        _zero_ref(stg, RPT)
        pltpu.sync_copy(stg, acc_sh.at[pl.ds(rbase, RPT)])
        _zero_ref(onesv, EBLK)
        _add_const(onesv, EBLK, 1.0)
        plsc.subcore_barrier()

        def blk(b, _):
            pltpu.sync_copy(colp_hbm.at[pl.ds(base + b * EBLK, EBLK)], cidx)
            pltpu.sync_copy(onesv, acc_sh.at[cidx], add=True)
            return 0
        lax.fori_loop(0, nblk, blk, 0)

        plsc.subcore_barrier()
        pltpu.sync_copy(acc_sh.at[pl.ds(rbase, RPT)], stg)
        pltpu.sync_copy(stg, out_hbm.at[pl.ds(cid * NP + rbase, RPT)])

    return deg_kernel


def _build_den(etp):
    epw = etp // NWK
    nblk = epw // EBLK
    mesh = plsc.VectorSubcoreMesh(core_axis_name="c", subcore_axis_name="s")

    @functools.partial(
        pl.kernel, mesh=mesh,
        compiler_params=pltpu.CompilerParams(needs_layout_passes=False),
        out_type=jax.ShapeDtypeStruct((NCC * HH * NP,), _F32),
        scratch_types=[
            pltpu.VMEM((NP,), _F32),
            pltpu.VMEM((NP,), _F32),
            pltpu.VMEM((EBLK,), jnp.int32),
            pltpu.VMEM((EBLK,), jnp.int32),
            pltpu.VMEM((EBLK,), _F32),
            pltpu.VMEM((RPT,), _F32),
            pltpu.VMEM_SHARED((NP,), _F32),
        ],
    )
    def den_kernel(ast_hbm, adt_hbm, rowp_hbm, colp_hbm, out_hbm,
                   asv, adv, ridx, cidx, exv, stg, acc_sh):
        cid = lax.axis_index("c")
        sid = lax.axis_index("s")
        base = (cid * NSS + sid) * epw
        rbase = sid * RPT

        for k in range(HH):
            pltpu.sync_copy(ast_hbm.at[pl.ds(k * NP, NP)], asv)
            pltpu.sync_copy(adt_hbm.at[pl.ds(k * NP, NP)], adv)
            _zero_ref(stg, RPT)
            pltpu.sync_copy(stg, acc_sh.at[pl.ds(rbase, RPT)])
            plsc.subcore_barrier()

            def blk(b, _):
                pltpu.sync_copy(rowp_hbm.at[pl.ds(base + b * EBLK, EBLK)], ridx)
                pltpu.sync_copy(colp_hbm.at[pl.ds(base + b * EBLK, EBLK)], cidx)
                def grp(j, _):
                    gsl = pl.ds(j * 16, 16)
                    a16 = (plsc.load_gather(asv, [ridx[gsl]]) +
                           plsc.load_gather(adv, [cidx[gsl]]))
                    a16 = jnp.where(a16 > 0, a16, 0.2 * a16)
                    exv[gsl] = jnp.exp(a16)
                    return 0
                lax.fori_loop(0, EBLK // 16, grp, 0)
                pltpu.sync_copy(exv, acc_sh.at[cidx], add=True)
                return 0
            lax.fori_loop(0, nblk, blk, 0)

            plsc.subcore_barrier()
            pltpu.sync_copy(acc_sh.at[pl.ds(rbase, RPT)], stg)
            pltpu.sync_copy(
                stg, out_hbm.at[pl.ds((cid * HH + k) * NP + rbase, RPT)])
            plsc.subcore_barrier()

    return den_kernel


def _build_wts_gcn(etp):
    epw = etp // NWK
    nblk = epw // EBLK
    mesh = plsc.VectorSubcoreMesh(core_axis_name="c", subcore_axis_name="s")

    @functools.partial(
        pl.kernel, mesh=mesh,
        compiler_params=pltpu.CompilerParams(needs_layout_passes=False),
        out_type=jax.ShapeDtypeStruct((etp,), _F32),
        scratch_types=[
            pltpu.VMEM((NP,), _F32),
            pltpu.VMEM((EBLK,), jnp.int32),
            pltpu.VMEM((EBLK,), jnp.int32),
            pltpu.VMEM((EBLK,), _F32),
        ],
    )
    def wts_kernel(dis_hbm, rowp_hbm, colp_hbm, out_hbm, disv, ridx, cidx, wb):
        cid = lax.axis_index("c")
        sid = lax.axis_index("s")
        base = (cid * NSS + sid) * epw
        pltpu.sync_copy(dis_hbm, disv)

        def blk(b, _):
            eb = base + b * EBLK
            pltpu.sync_copy(rowp_hbm.at[pl.ds(eb, EBLK)], ridx)
            pltpu.sync_copy(colp_hbm.at[pl.ds(eb, EBLK)], cidx)
            def grp(j, _):
                gsl = pl.ds(j * 16, 16)
                wb[gsl] = (plsc.load_gather(disv, [ridx[gsl]]) *
                           plsc.load_gather(disv, [cidx[gsl]]))
                return 0
            lax.fori_loop(0, EBLK // 16, grp, 0)
            pltpu.sync_copy(wb, out_hbm.at[pl.ds(eb, EBLK)])
            return 0
        lax.fori_loop(0, nblk, blk, 0)

    return wts_kernel


def _build_wts_gat(etp):
    epw = etp // NWK
    nblk = epw // EBLK
    mesh = plsc.VectorSubcoreMesh(core_axis_name="c", subcore_axis_name="s")

    @functools.partial(
        pl.kernel, mesh=mesh,
        compiler_params=pltpu.CompilerParams(needs_layout_passes=False),
        out_type=jax.ShapeDtypeStruct((HH * etp,), _F32),
        scratch_types=[
            pltpu.VMEM((NP,), _F32),
            pltpu.VMEM((NP,), _F32),
            pltpu.VMEM((NP,), _F32),
            pltpu.VMEM((EBLK,), jnp.int32),
            pltpu.VMEM((EBLK,), jnp.int32),
            pltpu.VMEM((EBLK,), _F32),
        ],
    )
    def wts_kernel(ast_hbm, adt_hbm, den_hbm, rowp_hbm, colp_hbm, out_hbm,
                   asv, adv, denv, ridx, cidx, wb):
        cid = lax.axis_index("c")
        sid = lax.axis_index("s")
        base = (cid * NSS + sid) * epw

        for k in range(HH):
            pltpu.sync_copy(ast_hbm.at[pl.ds(k * NP, NP)], asv)
            pltpu.sync_copy(adt_hbm.at[pl.ds(k * NP, NP)], adv)
            pltpu.sync_copy(den_hbm.at[pl.ds(k * NP, NP)], denv)

            def blk(b, _):
                eb = base + b * EBLK
                pltpu.sync_copy(rowp_hbm.at[pl.ds(eb, EBLK)], ridx)
                pltpu.sync_copy(colp_hbm.at[pl.ds(eb, EBLK)], cidx)
                def grp(j, _):
                    gsl = pl.ds(j * 16, 16)
                    r16 = ridx[gsl]
                    c16 = cidx[gsl]
                    a16 = (plsc.load_gather(asv, [r16]) +
                           plsc.load_gather(adv, [c16]))
                    a16 = jnp.where(a16 > 0, a16, 0.2 * a16)
                    ex = jnp.exp(a16)
                    den16 = plsc.load_gather(denv, [c16])
                    wb[gsl] = ex / (den16 + 1e-16)
                    return 0
                lax.fori_loop(0, EBLK // 16, grp, 0)
                pltpu.sync_copy(wb, out_hbm.at[pl.ds(k * etp + eb, EBLK)])
                return 0
            lax.fori_loop(0, nblk, blk, 0)

    return wts_kernel


def _build_prop(etp, mode):
    d = 128
    epw = etp // NWK
    nblk = epw // EBLK
    assert nblk % 2 == 0 and nblk >= 2
    mesh = plsc.VectorSubcoreMesh(core_axis_name="c", subcore_axis_name="s")

    if mode == "gcn":
        out_type = jax.ShapeDtypeStruct((NCC, NP, d), _F32)
        wshape = (etp,)
    else:
        out_type = jax.ShapeDtypeStruct((HH, NCC, NP, d), _F32)
        wshape = (HH * etp,)

    @functools.partial(
        pl.kernel, mesh=mesh,
        compiler_params=pltpu.CompilerParams(needs_layout_passes=False),
        out_type=out_type,
        scratch_types=[
            pltpu.VMEM((EBLK, d), _F32),
            pltpu.VMEM((EBLK, d), _F32),
            pltpu.VMEM((CH, d), _F32),
            pltpu.VMEM((EBLK,), jnp.int32),
            pltpu.VMEM((EBLK,), jnp.int32),
            pltpu.VMEM((EBLK,), jnp.int32),
            pltpu.VMEM((EBLK,), jnp.int32),
            pltpu.VMEM((EBLK,), _F32),
            pltpu.VMEM((EBLK,), _F32),
            pltpu.VMEM_SHARED((NP, d), _F32),
            pltpu.SemaphoreType.DMA,
            pltpu.SemaphoreType.DMA,
        ],
    )
    def prop_kernel(tbl_hbm, w_hbm, rowp_hbm, colp_hbm, out_hbm,
                    rows_a, rows_b, stg, ridx_a, ridx_b, cidx_a, cidx_b,
                    wv_a, wv_b, acc_sh, sem_a, sem_b):
        cid = lax.axis_index("c")
        sid = lax.axis_index("s")
        base = (cid * NSS + sid) * epw
        rbase = sid * RPT
        bufs = ((rows_a, ridx_a, cidx_a, wv_a, sem_a),
                (rows_b, ridx_b, cidx_b, wv_b, sem_b))

        def one_pass(woff, out_at):
            _zero_2d(stg, CH, d)
            for i in range(RPT // CH):
                pltpu.sync_copy(stg, acc_sh.at[pl.ds(rbase + i * CH, CH)])
            plsc.subcore_barrier()

            def issue(b, t):
                eb = base + b * EBLK
                pltpu.sync_copy(rowp_hbm.at[pl.ds(eb, EBLK)], t[1])
                pltpu.sync_copy(colp_hbm.at[pl.ds(eb, EBLK)], t[2])
                pltpu.sync_copy(w_hbm.at[pl.ds(woff + eb, EBLK)], t[3])
                pltpu.make_async_copy(tbl_hbm.at[t[1]], t[0], t[4]).start()

            def wait(t):
                pltpu.make_async_copy(tbl_hbm.at[t[1]], t[0], t[4]).wait()

            def crunch(t):
                def mgrp(j, _):
                    w16 = t[3][pl.ds(j * 16, 16)]
                    for r in range(16):
                        wb = _bcast(w16, r)
                        ri = j * 16 + r
                        for c in range(d // 16):
                            csl = pl.ds(c * 16, 16)
                            t[0][ri, csl] = t[0][ri, csl] * wb
                    return 0
                lax.fori_loop(0, EBLK // 16, mgrp, 0)
                pltpu.sync_copy(t[0], acc_sh.at[t[2]], add=True)

            issue(0, bufs[0])

            def body(i, _):
                b = 2 * i
                issue(b + 1, bufs[1])
                wait(bufs[0]); crunch(bufs[0])
                issue(b + 2, bufs[0])
                wait(bufs[1]); crunch(bufs[1])
                return 0
            lax.fori_loop(0, nblk // 2 - 1, body, 0)
            issue(nblk - 1, bufs[1])
            wait(bufs[0]); crunch(bufs[0])
            wait(bufs[1]); crunch(bufs[1])

            plsc.subcore_barrier()
            for i in range(RPT // CH):
                dsl = pl.ds(rbase + i * CH, CH)
                pltpu.sync_copy(acc_sh.at[dsl], stg)
                pltpu.sync_copy(stg, out_at(dsl))
            plsc.subcore_barrier()

        if mode == "gcn":
            one_pass(0, lambda dsl: out_hbm.at[cid, dsl])
        else:
            for k in range(HH):
                one_pass(k * etp, lambda dsl, k=k: out_hbm.at[k, cid, dsl])

    return prop_kernel


def _gelu(t):
    return 0.5 * t * (1.0 + lax.erf(t * (2.0 ** -0.5)))


def _bn_rows(h, g, b):
    mu = jnp.mean(h, axis=0)
    var = jnp.mean((h - mu) ** 2, axis=0)
    return (h - mu) / jnp.sqrt(var + 1e-5) * g + b


def _t_dis(degp):
    def body(degp_ref, dis_ref):
        deg = degp_ref[0, :] + degp_ref[1, :]
        dis_ref[...] = jnp.where(deg > 0, lax.rsqrt(deg), 0.0)
    return pl.pallas_call(
        body, out_shape=jax.ShapeDtypeStruct((NP,), _F32))(degp)


def _t0(x, w1):
    def body(x_ref, w_ref, o_ref):
        o_ref[...] = jnp.dot(x_ref[...], w_ref[...],
                             preferred_element_type=_F32)
    return pl.pallas_call(
        body, out_shape=jax.ShapeDtypeStruct((NN, 128), _F32))(x, w1)


def _t_den(denp):
    def body(dp_ref, d_ref):
        d_ref[...] = dp_ref[0, :] + dp_ref[1, :]
    return pl.pallas_call(
        body, out_shape=jax.ShapeDtypeStruct((HH * NP,), _F32))(denp)


def _t1(p1, b1, g1, be1):
    def body(p_ref, b_ref, g_ref, be_ref, h_ref):
        hh = p_ref[0, :NN, :] + p_ref[1, :NN, :] + b_ref[...]
        h_ref[...] = _gelu(_bn_rows(hh, g_ref[...], be_ref[...]))
    return pl.pallas_call(
        body, out_shape=jax.ShapeDtypeStruct((NN, 128), _F32))(
            p1, b1, g1, be1)


def _t2(p2, wh, bh, gh, beh, h1, wg3, ats, atd):
    def body(p_ref, w_ref, b_ref, g_ref, be_ref, h1_ref, wg_ref, ats_ref,
             atd_ref, h2_ref, ast_ref, adt_ref):
        xx = p_ref[0, :NN, :] + p_ref[1, :NN, :]
        hh = jnp.dot(xx, w_ref[...], preferred_element_type=_F32,
                     precision=_HI) + b_ref[...]
        h2 = _gelu(_bn_rows(hh, g_ref[...], be_ref[...])) + h1_ref[...]
        h2_ref[...] = h2
        h2r = h2.astype(jnp.bfloat16).astype(_F32)
        va_s = jnp.sum(wg_ref[...] * ats_ref[...][None, :, :], axis=-1)
        va_d = jnp.sum(wg_ref[...] * atd_ref[...][None, :, :], axis=-1)
        a_sT = jax.lax.dot_general(va_s, h2r, (((0,), (1,)), ((), ())),
                                   preferred_element_type=_F32,
                                   precision=_HI)
        a_dT = jax.lax.dot_general(va_d, h2r, (((0,), (1,)), ((), ())),
                                   preferred_element_type=_F32,
                                   precision=_HI)
        pad = jnp.zeros((HH, NP - NN), _F32)
        ast_ref[...] = jnp.concatenate([a_sT, pad], axis=1)
        adt_ref[...] = jnp.concatenate([a_dT, pad], axis=1)
    return pl.pallas_call(
        body,
        out_shape=[jax.ShapeDtypeStruct((NN, 128), _F32),
                   jax.ShapeDtypeStruct((HH, NP), _F32),
                   jax.ShapeDtypeStruct((HH, NP), _F32)])(
            p2, wh, bh, gh, beh, h1, wg3, ats, atd)


def _t4(pg, wgh, bg):
    def body(p_ref, w_ref, bg_ref, g_ref):
        k = pl.program_id(0)
        pk = p_ref[0, 0, :NN, :] + p_ref[0, 1, :NN, :]
        contrib = jnp.dot(pk, w_ref[0], preferred_element_type=_F32,
                          precision=_HI) * (1.0 / HH)
        @pl.when(k == 0)
        def _():
            g_ref[...] = contrib + bg_ref[...]
        @pl.when(k > 0)
        def _():
            g_ref[...] = g_ref[...] + contrib
    return pl.pallas_call(
        body,
        grid=(HH,),
        in_specs=[pl.BlockSpec((1, NCC, NP, 128), lambda k: (k, 0, 0, 0)),
                  pl.BlockSpec((1, 128, 128), lambda k: (k, 0, 0)),
                  pl.BlockSpec((128,), lambda k: (0,))],
        out_specs=pl.BlockSpec((NN, 128), lambda k: (0, 0)),
        out_shape=jax.ShapeDtypeStruct((NN, 128), _F32))(pg, wgh, bg)


def _t5(p4, wo, bo, go, beo, batchf):
    def body(p_ref, w_ref, b_ref, g_ref, be_ref, bt_ref, s_ref, st_ref):
        xx = p_ref[0, :NN, :] + p_ref[1, :NN, :]
        hh = jnp.dot(xx, w_ref[...], preferred_element_type=_F32,
                     precision=_HI) + b_ref[...]
        hj = _gelu(_bn_rows(hh, g_ref[...], be_ref[...]))
        oh = (bt_ref[...] == lax.broadcasted_iota(jnp.int32, (NN, NB), 1)
              .astype(_F32)).astype(_F32)
        s_ref[...] = jax.lax.dot_general(oh, hj, (((0,), (0,)), ((), ())),
                                         preferred_element_type=_F32,
                                         precision=_HI)
        lane = lax.broadcasted_iota(jnp.int32, (1, 1, 128), 2)
        stats = jnp.where(lane == 0, jnp.sum(hj),
                          jnp.where(lane == 1, jnp.sum(hj * hj), 0.0))
        st_ref[...] = stats
    return pl.pallas_call(
        body,
        grid=(4,),
        in_specs=[pl.BlockSpec((NCC, NP, 128), lambda j: (0, 0, 0)),
                  pl.BlockSpec((128, 128), lambda j: (0, j)),
                  pl.BlockSpec((128,), lambda j: (j,)),
                  pl.BlockSpec((128,), lambda j: (j,)),
                  pl.BlockSpec((128,), lambda j: (j,)),
                  pl.BlockSpec((NN, 1), lambda j: (0, 0))],
        out_specs=[pl.BlockSpec((NB, 128), lambda j: (0, j)),
                   pl.BlockSpec((1, 1, 128), lambda j: (j, 0, 0))],
        out_shape=[jax.ShapeDtypeStruct((NB, 512), _F32),
                   jax.ShapeDtypeStruct((4, 1, 128), _F32)])(
            p4, wo, bo, go, beo, batchf)


def _t6(s, stats, batchf, lnw, lnb, pw, pb, png, pnb, fw, fb, l2w, l2b, cw, cb):
    def body(s_ref, st_ref, bt_ref, lnw_ref, lnb_ref, pw_ref, pb_ref, png_ref,
             pnb_ref, fw_ref, fb_ref, l2w_ref, l2b_ref, cw_ref, cb_ref,
             xn_ref, c_ref):
        oh = (bt_ref[...] == lax.broadcasted_iota(jnp.int32, (NN, NB), 1)
              .astype(_F32)).astype(_F32)
        cnt = jnp.sum(oh, axis=0)[:, None]
        g = s_ref[...] / jnp.maximum(cnt, 1.0)
        tot = float(NN * 512)
        musum = jnp.sum(st_ref[:, 0, 0])
        sqsum = jnp.sum(st_ref[:, 0, 1])
        mu = musum / tot
        var = sqsum / tot - mu * mu
        g = (g - mu) / jnp.sqrt(var + 1e-5) * lnw_ref[...] + lnb_ref[...]
        p = jnp.dot(g, pw_ref[...], preferred_element_type=_F32) + pb_ref[...]
        p = _gelu(_bn_rows(p, png_ref[...], pnb_ref[...]))
        q = jnp.dot(p, fw_ref[...], preferred_element_type=_F32) + fb_ref[...] + p
        mu2 = jnp.mean(q, axis=-1, keepdims=True)
        var2 = jnp.mean((q - mu2) ** 2, axis=-1, keepdims=True)
        z = (q - mu2) / jnp.sqrt(var2 + 1e-5) * l2w_ref[...] + l2b_ref[...]
        nrm = jnp.sqrt(jnp.sum(z * z, axis=1, keepdims=True))
        xn = z / jnp.maximum(nrm, 1e-12)
        xn_ref[...] = xn
        lg = jnp.dot(xn, cw_ref[...], preferred_element_type=_F32) + cb_ref[...]
        m = jnp.max(lg, axis=1, keepdims=True)
        lse = m + jnp.log(jnp.sum(jnp.exp(lg - m), axis=1, keepdims=True))
        c_ref[...] = lg - lse
    return pl.pallas_call(
        body,
        out_shape=[jax.ShapeDtypeStruct((NB, 128), _F32),
                   jax.ShapeDtypeStruct((NB, 10), _F32)])(
            s, stats, batchf, lnw, lnb, pw, pb, png, pnb, fw, fb, l2w, l2b,
            cw, cb)


def kernel(x, W1, b1, g1, be1, Wh, bh, gh, beh, Wg, att_s, att_d, bg, Wo, bo,
           go, beo, lnw, lnb, pW, pb, png, pnb, fW, fb, l2w, l2b, cW, cb,
           edge_index, batch):
    n = x.shape[0]
    e = edge_index.shape[1]
    et = e + n
    ealign = NWK * EBLK
    etp = ((et + ealign - 1) // ealign) * ealign

    sl = jnp.arange(n, dtype=edge_index.dtype)
    rowp = jnp.pad(jnp.concatenate([edge_index[0], sl]), (0, etp - et))
    colp = jnp.pad(jnp.concatenate([edge_index[1], sl]), (0, etp - et),
                   constant_values=n)

    wg3_r = _rnd(Wg.reshape(128, HH, 128))
    wgh_r = jnp.transpose(wg3_r, (1, 0, 2))
    wh_r = _rnd(Wh)
    wo_r = _rnd(Wo)
    batchf = batch.astype(_F32)[:, None]

    degp = _build_deg(etp)(colp).reshape(NCC, NP)
    dis = _t_dis(degp)

    wgcn = _build_wts_gcn(etp)(dis, rowp, colp)

    x1 = _t0(x, W1)
    p1 = _build_prop(etp, "gcn")(x1, wgcn, rowp, colp)
    h1 = _t1(p1, b1, g1, be1)

    h1r = _rnd(h1)
    p2 = _build_prop(etp, "gcn")(h1r, wgcn, rowp, colp)
    h2, a_sT, a_dT = _t2(p2, wh_r, bh, gh, beh, h1, wg3_r, att_s, att_d)

    astf = a_sT.reshape(HH * NP)
    adtf = a_dT.reshape(HH * NP)
    denp = _build_den(etp)(astf, adtf, rowp, colp)
    den = _t_den(denp.reshape(NCC, HH * NP))
    wgat = _build_wts_gat(etp)(astf, adtf, den, rowp, colp)
    h2r = _rnd(h2)
    pg = _build_prop(etp, "gat")(h2r, wgat, rowp, colp)
    g = _t4(pg, wgh_r, bg)

    gr = _rnd(g)
    p4 = _build_prop(etp, "gcn")(gr, wgcn, rowp, colp)
    s, stats = _t5(p4, wo_r, bo, go, beo, batchf)
    xn, c = _t6(s, stats, batchf, lnw, lnb, pW, pb, png, pnb, fW, fb,
                l2w, l2b, cW, cb)
    return xn, c


# speedup vs baseline: 13.2847x; 1.0005x over previous
import functools

import jax
import jax.numpy as jnp
from jax import lax
from jax.experimental import pallas as pl
from jax.experimental.pallas import tpu as pltpu
from jax.experimental.pallas import tpu_sc as plsc

NN = 10000
NP = 10240
HH = 5
NB = 64
EBLK = 128
NCC = 2
NSS = 16
NWK = NCC * NSS
RPT = NP // NSS
CH = 80

_F32 = jnp.float32
_HI = lax.Precision.HIGHEST


def _rnd(a):
    return a.astype(jnp.bfloat16).astype(_F32)


_GDN = lax.GatherDimensionNumbers(offset_dims=(), collapsed_slice_dims=(0,),
                                  start_index_map=(0,))


def _bcast(v16, r):
    idx = jnp.full((16, 1), r, jnp.int32)
    return lax.gather(v16, idx, _GDN, (1,),
                      mode=lax.GatherScatterMode.PROMISE_IN_BOUNDS)


def _zero_ref(ref, nwords):
    def body(i, _):
        ref[pl.ds(i * 16, 16)] = jnp.zeros((16,), _F32)
        return 0
    lax.fori_loop(0, nwords // 16, body, 0)


def _zero_2d(ref, rows, cols):
    def body(i, _):
        for c in range(cols // 16):
            ref[i, pl.ds(c * 16, 16)] = jnp.zeros((16,), _F32)
        return 0
    lax.fori_loop(0, rows, body, 0)


def _add_const(ref, nwords, val):
    def body(i, _):
        sl = pl.ds(i * 16, 16)
        ref[sl] = ref[sl] + val
        return 0
    lax.fori_loop(0, nwords // 16, body, 0)


def _build_deg(etp):
    epw = etp // NWK
    nblk = epw // EBLK
    mesh = plsc.VectorSubcoreMesh(core_axis_name="c", subcore_axis_name="s")

    @functools.partial(
        pl.kernel, mesh=mesh,
        compiler_params=pltpu.CompilerParams(needs_layout_passes=False),
        out_type=jax.ShapeDtypeStruct((NCC * NP,), _F32),
        scratch_types=[
            pltpu.VMEM((EBLK,), jnp.int32),
            pltpu.VMEM((EBLK,), _F32),
            pltpu.VMEM((RPT,), _F32),
            pltpu.VMEM_SHARED((NP,), _F32),
        ],
    )
    def deg_kernel(colp_hbm, out_hbm, cidx, onesv, stg, acc_sh):
        cid = lax.axis_index("c")
        sid = lax.axis_index("s")
        base = (cid * NSS + sid) * epw
        rbase = sid * RPT
        _zero_ref(stg, RPT)
        pltpu.sync_copy(stg, acc_sh.at[pl.ds(rbase, RPT)])
        _zero_ref(onesv, EBLK)
        _add_const(onesv, EBLK, 1.0)
        plsc.subcore_barrier()

        def blk(b, _):
            pltpu.sync_copy(colp_hbm.at[pl.ds(base + b * EBLK, EBLK)], cidx)
            pltpu.sync_copy(onesv, acc_sh.at[cidx], add=True)
            return 0
        lax.fori_loop(0, nblk, blk, 0)

        plsc.subcore_barrier()
        pltpu.sync_copy(acc_sh.at[pl.ds(rbase, RPT)], stg)
        pltpu.sync_copy(stg, out_hbm.at[pl.ds(cid * NP + rbase, RPT)])

    return deg_kernel


def _build_den(etp):
    epw = etp // NWK
    nblk = epw // EBLK
    mesh = plsc.VectorSubcoreMesh(core_axis_name="c", subcore_axis_name="s")

    @functools.partial(
        pl.kernel, mesh=mesh,
        compiler_params=pltpu.CompilerParams(needs_layout_passes=False),
        out_type=jax.ShapeDtypeStruct((NCC * HH * NP,), _F32),
        scratch_types=[
            pltpu.VMEM((NP,), _F32),
            pltpu.VMEM((NP,), _F32),
            pltpu.VMEM((EBLK,), jnp.int32),
            pltpu.VMEM((EBLK,), jnp.int32),
            pltpu.VMEM((EBLK,), _F32),
            pltpu.VMEM((RPT,), _F32),
            pltpu.VMEM_SHARED((NP,), _F32),
        ],
    )
    def den_kernel(ast_hbm, adt_hbm, rowp_hbm, colp_hbm, out_hbm,
                   asv, adv, ridx, cidx, exv, stg, acc_sh):
        cid = lax.axis_index("c")
        sid = lax.axis_index("s")
        base = (cid * NSS + sid) * epw
        rbase = sid * RPT

        for k in range(HH):
            pltpu.sync_copy(ast_hbm.at[pl.ds(k * NP, NP)], asv)
            pltpu.sync_copy(adt_hbm.at[pl.ds(k * NP, NP)], adv)
            _zero_ref(stg, RPT)
            pltpu.sync_copy(stg, acc_sh.at[pl.ds(rbase, RPT)])
            plsc.subcore_barrier()

            def blk(b, _):
                pltpu.sync_copy(rowp_hbm.at[pl.ds(base + b * EBLK, EBLK)], ridx)
                pltpu.sync_copy(colp_hbm.at[pl.ds(base + b * EBLK, EBLK)], cidx)
                def grp(j, _):
                    gsl = pl.ds(j * 16, 16)
                    a16 = (plsc.load_gather(asv, [ridx[gsl]]) +
                           plsc.load_gather(adv, [cidx[gsl]]))
                    a16 = jnp.where(a16 > 0, a16, 0.2 * a16)
                    exv[gsl] = jnp.exp(a16)
                    return 0
                lax.fori_loop(0, EBLK // 16, grp, 0)
                pltpu.sync_copy(exv, acc_sh.at[cidx], add=True)
                return 0
            lax.fori_loop(0, nblk, blk, 0)

            plsc.subcore_barrier()
            pltpu.sync_copy(acc_sh.at[pl.ds(rbase, RPT)], stg)
            pltpu.sync_copy(
                stg, out_hbm.at[pl.ds((cid * HH + k) * NP + rbase, RPT)])
            plsc.subcore_barrier()

    return den_kernel


def _build_wts_gcn(etp):
    epw = etp // NWK
    nblk = epw // EBLK
    mesh = plsc.VectorSubcoreMesh(core_axis_name="c", subcore_axis_name="s")

    @functools.partial(
        pl.kernel, mesh=mesh,
        compiler_params=pltpu.CompilerParams(needs_layout_passes=False),
        out_type=jax.ShapeDtypeStruct((etp,), _F32),
        scratch_types=[
            pltpu.VMEM((NP,), _F32),
            pltpu.VMEM((EBLK,), jnp.int32),
            pltpu.VMEM((EBLK,), jnp.int32),
            pltpu.VMEM((EBLK,), _F32),
        ],
    )
    def wts_kernel(dis_hbm, rowp_hbm, colp_hbm, out_hbm, disv, ridx, cidx, wb):
        cid = lax.axis_index("c")
        sid = lax.axis_index("s")
        base = (cid * NSS + sid) * epw
        pltpu.sync_copy(dis_hbm, disv)

        def blk(b, _):
            eb = base + b * EBLK
            pltpu.sync_copy(rowp_hbm.at[pl.ds(eb, EBLK)], ridx)
            pltpu.sync_copy(colp_hbm.at[pl.ds(eb, EBLK)], cidx)
            def grp(j, _):
                gsl = pl.ds(j * 16, 16)
                wb[gsl] = (plsc.load_gather(disv, [ridx[gsl]]) *
                           plsc.load_gather(disv, [cidx[gsl]]))
                return 0
            lax.fori_loop(0, EBLK // 16, grp, 0)
            pltpu.sync_copy(wb, out_hbm.at[pl.ds(eb, EBLK)])
            return 0
        lax.fori_loop(0, nblk, blk, 0)

    return wts_kernel


def _build_wts_gat(etp):
    epw = etp // NWK
    nblk = epw // EBLK
    mesh = plsc.VectorSubcoreMesh(core_axis_name="c", subcore_axis_name="s")

    @functools.partial(
        pl.kernel, mesh=mesh,
        compiler_params=pltpu.CompilerParams(needs_layout_passes=False),
        out_type=jax.ShapeDtypeStruct((HH * etp,), _F32),
        scratch_types=[
            pltpu.VMEM((NP,), _F32),
            pltpu.VMEM((NP,), _F32),
            pltpu.VMEM((NP,), _F32),
            pltpu.VMEM((EBLK,), jnp.int32),
            pltpu.VMEM((EBLK,), jnp.int32),
            pltpu.VMEM((EBLK,), _F32),
        ],
    )
    def wts_kernel(ast_hbm, adt_hbm, den_hbm, rowp_hbm, colp_hbm, out_hbm,
                   asv, adv, denv, ridx, cidx, wb):
        cid = lax.axis_index("c")
        sid = lax.axis_index("s")
        base = (cid * NSS + sid) * epw

        for k in range(HH):
            pltpu.sync_copy(ast_hbm.at[pl.ds(k * NP, NP)], asv)
            pltpu.sync_copy(adt_hbm.at[pl.ds(k * NP, NP)], adv)
            pltpu.sync_copy(den_hbm.at[pl.ds(k * NP, NP)], denv)

            def blk(b, _):
                eb = base + b * EBLK
                pltpu.sync_copy(rowp_hbm.at[pl.ds(eb, EBLK)], ridx)
                pltpu.sync_copy(colp_hbm.at[pl.ds(eb, EBLK)], cidx)
                def grp(j, _):
                    gsl = pl.ds(j * 16, 16)
                    r16 = ridx[gsl]
                    c16 = cidx[gsl]
                    a16 = (plsc.load_gather(asv, [r16]) +
                           plsc.load_gather(adv, [c16]))
                    a16 = jnp.where(a16 > 0, a16, 0.2 * a16)
                    ex = jnp.exp(a16)
                    den16 = plsc.load_gather(denv, [c16])
                    wb[gsl] = ex / (den16 + 1e-16)
                    return 0
                lax.fori_loop(0, EBLK // 16, grp, 0)
                pltpu.sync_copy(wb, out_hbm.at[pl.ds(k * etp + eb, EBLK)])
                return 0
            lax.fori_loop(0, nblk, blk, 0)

    return wts_kernel


def _build_prop(etp, mode):
    d = 128
    epw = etp // NWK
    nblk = epw // EBLK
    assert nblk % 2 == 0 and nblk >= 2
    mesh = plsc.VectorSubcoreMesh(core_axis_name="c", subcore_axis_name="s")

    if mode == "gcn":
        out_type = jax.ShapeDtypeStruct((NCC, NP, d), _F32)
        wshape = (etp,)
    else:
        out_type = jax.ShapeDtypeStruct((HH, NCC, NP, d), _F32)
        wshape = (HH * etp,)

    @functools.partial(
        pl.kernel, mesh=mesh,
        compiler_params=pltpu.CompilerParams(needs_layout_passes=False),
        out_type=out_type,
        scratch_types=[
            pltpu.VMEM((EBLK, d), _F32),
            pltpu.VMEM((EBLK, d), _F32),
            pltpu.VMEM((CH, d), _F32),
            pltpu.VMEM((EBLK,), jnp.int32),
            pltpu.VMEM((EBLK,), jnp.int32),
            pltpu.VMEM((EBLK,), jnp.int32),
            pltpu.VMEM((EBLK,), jnp.int32),
            pltpu.VMEM((EBLK,), _F32),
            pltpu.VMEM((EBLK,), _F32),
            pltpu.VMEM_SHARED((NP, d), _F32),
            pltpu.SemaphoreType.DMA,
            pltpu.SemaphoreType.DMA,
        ],
    )
    def prop_kernel(tbl_hbm, w_hbm, rowp_hbm, colp_hbm, out_hbm,
                    rows_a, rows_b, stg, ridx_a, ridx_b, cidx_a, cidx_b,
                    wv_a, wv_b, acc_sh, sem_a, sem_b):
        cid = lax.axis_index("c")
        sid = lax.axis_index("s")
        base = (cid * NSS + sid) * epw
        rbase = sid * RPT
        bufs = ((rows_a, ridx_a, cidx_a, wv_a, sem_a),
                (rows_b, ridx_b, cidx_b, wv_b, sem_b))

        def one_pass(woff, out_at):
            _zero_2d(stg, CH, d)
            for i in range(RPT // CH):
                pltpu.sync_copy(stg, acc_sh.at[pl.ds(rbase + i * CH, CH)])
            plsc.subcore_barrier()

            def issue(b, t):
                eb = base + b * EBLK
                pltpu.sync_copy(rowp_hbm.at[pl.ds(eb, EBLK)], t[1])
                pltpu.sync_copy(colp_hbm.at[pl.ds(eb, EBLK)], t[2])
                pltpu.sync_copy(w_hbm.at[pl.ds(woff + eb, EBLK)], t[3])
                pltpu.make_async_copy(tbl_hbm.at[t[1]], t[0], t[4]).start()

            def wait(t):
                pltpu.make_async_copy(tbl_hbm.at[t[1]], t[0], t[4]).wait()

            def crunch(t):
                def mgrp(j, _):
                    w16 = t[3][pl.ds(j * 16, 16)]
                    for r in range(16):
                        wb = _bcast(w16, r)
                        ri = j * 16 + r
                        for c in range(d // 16):
                            csl = pl.ds(c * 16, 16)
                            t[0][ri, csl] = t[0][ri, csl] * wb
                    return 0
                lax.fori_loop(0, EBLK // 16, mgrp, 0)
                pltpu.sync_copy(t[0], acc_sh.at[t[2]], add=True)

            issue(0, bufs[0])

            def body(i, _):
                b = 2 * i
                issue(b + 1, bufs[1])
                wait(bufs[0]); crunch(bufs[0])
                issue(b + 2, bufs[0])
                wait(bufs[1]); crunch(bufs[1])
                return 0
            lax.fori_loop(0, nblk // 2 - 1, body, 0)
            issue(nblk - 1, bufs[1])
            wait(bufs[0]); crunch(bufs[0])
            wait(bufs[1]); crunch(bufs[1])

            plsc.subcore_barrier()
            for i in range(RPT // CH):
                dsl = pl.ds(rbase + i * CH, CH)
                pltpu.sync_copy(acc_sh.at[dsl], stg)
                pltpu.sync_copy(stg, out_at(dsl))
            plsc.subcore_barrier()

        if mode == "gcn":
            one_pass(0, lambda dsl: out_hbm.at[cid, dsl])
        else:
            for k in range(HH):
                one_pass(k * etp, lambda dsl, k=k: out_hbm.at[k, cid, dsl])

    return prop_kernel


def _gelu(t):
    return 0.5 * t * (1.0 + lax.erf(t * (2.0 ** -0.5)))


def _bn_rows(h, g, b):
    mu = jnp.mean(h, axis=0)
    var = jnp.mean((h - mu) ** 2, axis=0)
    return (h - mu) / jnp.sqrt(var + 1e-5) * g + b


def _t_dis(degp):
    def body(degp_ref, dis_ref):
        deg = degp_ref[0, :] + degp_ref[1, :]
        dis_ref[...] = jnp.where(deg > 0, lax.rsqrt(deg), 0.0)
    return pl.pallas_call(
        body, out_shape=jax.ShapeDtypeStruct((NP,), _F32))(degp)


def _t0(x, w1):
    def body(x_ref, w_ref, o_ref):
        o_ref[...] = jnp.dot(x_ref[...], w_ref[...],
                             preferred_element_type=_F32)
    return pl.pallas_call(
        body, out_shape=jax.ShapeDtypeStruct((NN, 128), _F32))(x, w1)


def _t_den(denp):
    def body(dp_ref, d_ref):
        d_ref[...] = dp_ref[0, :] + dp_ref[1, :]
    return pl.pallas_call(
        body, out_shape=jax.ShapeDtypeStruct((HH * NP,), _F32))(denp)


def _t1(p1, b1, g1, be1):
    def body(p_ref, b_ref, g_ref, be_ref, h_ref):
        hh = p_ref[0, :NN, :] + p_ref[1, :NN, :] + b_ref[...]
        h_ref[...] = _gelu(_bn_rows(hh, g_ref[...], be_ref[...]))
    return pl.pallas_call(
        body, out_shape=jax.ShapeDtypeStruct((NN, 128), _F32))(
            p1, b1, g1, be1)


def _t2(p2, wh, bh, gh, beh, h1, wg3, ats, atd):
    def body(p_ref, w_ref, b_ref, g_ref, be_ref, h1_ref, wg_ref, ats_ref,
             atd_ref, h2_ref, ast_ref, adt_ref):
        xx = p_ref[0, :NN, :] + p_ref[1, :NN, :]
        hh = jnp.dot(xx, w_ref[...], preferred_element_type=_F32,
                     precision=_HI) + b_ref[...]
        h2 = _gelu(_bn_rows(hh, g_ref[...], be_ref[...])) + h1_ref[...]
        h2_ref[...] = h2
        h2r = h2.astype(jnp.bfloat16).astype(_F32)
        va_s = jnp.sum(wg_ref[...] * ats_ref[...][None, :, :], axis=-1)
        va_d = jnp.sum(wg_ref[...] * atd_ref[...][None, :, :], axis=-1)
        a_sT = jax.lax.dot_general(va_s, h2r, (((0,), (1,)), ((), ())),
                                   preferred_element_type=_F32,
                                   precision=_HI)
        a_dT = jax.lax.dot_general(va_d, h2r, (((0,), (1,)), ((), ())),
                                   preferred_element_type=_F32,
                                   precision=_HI)
        pad = jnp.zeros((HH, NP - NN), _F32)
        ast_ref[...] = jnp.concatenate([a_sT, pad], axis=1)
        adt_ref[...] = jnp.concatenate([a_dT, pad], axis=1)
    return pl.pallas_call(
        body,
        out_shape=[jax.ShapeDtypeStruct((NN, 128), _F32),
                   jax.ShapeDtypeStruct((HH, NP), _F32),
                   jax.ShapeDtypeStruct((HH, NP), _F32)])(
            p2, wh, bh, gh, beh, h1, wg3, ats, atd)


def _t4(pg, wgh, bg):
    def body(p_ref, w_ref, bg_ref, g_ref):
        k = pl.program_id(0)
        pk = p_ref[0, 0, :NN, :] + p_ref[0, 1, :NN, :]
        contrib = jnp.dot(pk, w_ref[0], preferred_element_type=_F32,
                          precision=_HI) * (1.0 / HH)
        @pl.when(k == 0)
        def _():
            g_ref[...] = contrib + bg_ref[...]
        @pl.when(k > 0)
        def _():
            g_ref[...] = g_ref[...] + contrib
    return pl.pallas_call(
        body,
        grid=(HH,),
        in_specs=[pl.BlockSpec((1, NCC, NP, 128), lambda k: (k, 0, 0, 0)),
                  pl.BlockSpec((1, 128, 128), lambda k: (k, 0, 0)),
                  pl.BlockSpec((128,), lambda k: (0,))],
        out_specs=pl.BlockSpec((NN, 128), lambda k: (0, 0)),
        out_shape=jax.ShapeDtypeStruct((NN, 128), _F32))(pg, wgh, bg)


def _t5(p4, wo, bo, go, beo, batchf):
    def body(p_ref, w_ref, b_ref, g_ref, be_ref, bt_ref, s_ref, st_ref):
        xx = p_ref[0, :NN, :] + p_ref[1, :NN, :]
        hh = jnp.dot(xx, w_ref[...], preferred_element_type=_F32,
                     precision=_HI) + b_ref[...]
        hj = _gelu(_bn_rows(hh, g_ref[...], be_ref[...]))
        oh = (bt_ref[...] == lax.broadcasted_iota(jnp.int32, (NN, NB), 1)
              .astype(_F32)).astype(_F32)
        s_ref[...] = jax.lax.dot_general(oh, hj, (((0,), (0,)), ((), ())),
                                         preferred_element_type=_F32,
                                         precision=_HI)
        lane = lax.broadcasted_iota(jnp.int32, (1, 1, 128), 2)
        stats = jnp.where(lane == 0, jnp.sum(hj),
                          jnp.where(lane == 1, jnp.sum(hj * hj), 0.0))
        st_ref[...] = stats
    return pl.pallas_call(
        body,
        grid=(4,),
        in_specs=[pl.BlockSpec((NCC, NP, 128), lambda j: (0, 0, 0)),
                  pl.BlockSpec((128, 128), lambda j: (0, j)),
                  pl.BlockSpec((128,), lambda j: (j,)),
                  pl.BlockSpec((128,), lambda j: (j,)),
                  pl.BlockSpec((128,), lambda j: (j,)),
                  pl.BlockSpec((NN, 1), lambda j: (0, 0))],
        out_specs=[pl.BlockSpec((NB, 128), lambda j: (0, j)),
                   pl.BlockSpec((1, 1, 128), lambda j: (j, 0, 0))],
        out_shape=[jax.ShapeDtypeStruct((NB, 512), _F32),
                   jax.ShapeDtypeStruct((4, 1, 128), _F32)])(
            p4, wo, bo, go, beo, batchf)


def _t6(s, stats, batchf, lnw, lnb, pw, pb, png, pnb, fw, fb, l2w, l2b, cw, cb):
    def body(s_ref, st_ref, bt_ref, lnw_ref, lnb_ref, pw_ref, pb_ref, png_ref,
             pnb_ref, fw_ref, fb_ref, l2w_ref, l2b_ref, cw_ref, cb_ref,
             xn_ref, c_ref):
        oh = (bt_ref[...] == lax.broadcasted_iota(jnp.int32, (NN, NB), 1)
              .astype(_F32)).astype(_F32)
        cnt = jnp.sum(oh, axis=0)[:, None]
        g = s_ref[...] / jnp.maximum(cnt, 1.0)
        tot = float(NN * 512)
        musum = jnp.sum(st_ref[:, 0, 0])
        sqsum = jnp.sum(st_ref[:, 0, 1])
        mu = musum / tot
        var = sqsum / tot - mu * mu
        g = (g - mu) / jnp.sqrt(var + 1e-5) * lnw_ref[...] + lnb_ref[...]
        p = jnp.dot(g, pw_ref[...], preferred_element_type=_F32) + pb_ref[...]
        p = _gelu(_bn_rows(p, png_ref[...], pnb_ref[...]))
        q = jnp.dot(p, fw_ref[...], preferred_element_type=_F32) + fb_ref[...] + p
        mu2 = jnp.mean(q, axis=-1, keepdims=True)
        var2 = jnp.mean((q - mu2) ** 2, axis=-1, keepdims=True)
        z = (q - mu2) / jnp.sqrt(var2 + 1e-5) * l2w_ref[...] + l2b_ref[...]
        nrm = jnp.sqrt(jnp.sum(z * z, axis=1, keepdims=True))
        xn = z / jnp.maximum(nrm, 1e-12)
        xn_ref[...] = xn
        lg = jnp.dot(xn, cw_ref[...], preferred_element_type=_F32) + cb_ref[...]
        m = jnp.max(lg, axis=1, keepdims=True)
        lse = m + jnp.log(jnp.sum(jnp.exp(lg - m), axis=1, keepdims=True))
        c_ref[...] = lg - lse
    return pl.pallas_call(
        body,
        out_shape=[jax.ShapeDtypeStruct((NB, 128), _F32),
                   jax.ShapeDtypeStruct((NB, 10), _F32)])(
            s, stats, batchf, lnw, lnb, pw, pb, png, pnb, fw, fb, l2w, l2b,
            cw, cb)


def kernel(x, W1, b1, g1, be1, Wh, bh, gh, beh, Wg, att_s, att_d, bg, Wo, bo,
           go, beo, lnw, lnb, pW, pb, png, pnb, fW, fb, l2w, l2b, cW, cb,
           edge_index, batch):
    n = x.shape[0]
    e = edge_index.shape[1]
    et = e + n
    ealign = NWK * EBLK
    etp = ((et + ealign - 1) // ealign) * ealign

    sl = jnp.arange(n, dtype=edge_index.dtype)
    rowp = jnp.pad(jnp.concatenate([edge_index[0], sl]), (0, etp - et))
    colp = jnp.pad(jnp.concatenate([edge_index[1], sl]), (0, etp - et),
                   constant_values=n)

    wg3_r = _rnd(Wg.reshape(128, HH, 128))
    wgh_r = jnp.transpose(wg3_r, (1, 0, 2))
    wh_r = _rnd(Wh)
    wo_r = _rnd(Wo)
    batchf = batch.astype(_F32)[:, None]

    degp = _build_deg(etp)(colp).reshape(NCC, NP)
    dis = _t_dis(degp)

    wgcn = _build_wts_gcn(etp)(dis, rowp, colp)

    x1 = _t0(x, W1)
    p1 = _build_prop(etp, "gcn")(x1, wgcn, rowp, colp)
    h1 = _t1(p1, b1, g1, be1)

    h1r = _rnd(h1)
    p2 = _build_prop(etp, "gcn")(h1r, wgcn, rowp, colp)
    h2, a_sT, a_dT = _t2(p2, wh_r, bh, gh, beh, h1, wg3_r, att_s, att_d)

    astf = a_sT.reshape(HH * NP)
    adtf = a_dT.reshape(HH * NP)
    denp = _build_den(etp)(astf, adtf, rowp, colp)
    den = _t_den(denp.reshape(NCC, HH * NP))
    wgat = _build_wts_gat(etp)(astf, adtf, den, rowp, colp)
    h2r = _rnd(h2)
    pg = _build_prop(etp, "gat")(h2r, wgat, rowp, colp)
    g = _t4(pg, wgh_r, bg)

    gr = _rnd(g)
    p4 = _build_prop(etp, "gcn")(gr, wgcn, rowp, colp)
    s, stats = _t5(p4, wo_r, bo, go, beo, batchf)
    xn, c = _t6(s, stats, batchf, lnw, lnb, pW, pb, png, pnb, fW, fb,
                l2w, l2b, cW, cb)
    return xn, c


# speedup vs baseline: 14.2391x; 1.0718x over previous
import functools

import jax
import jax.numpy as jnp
from jax import lax
from jax.experimental import pallas as pl
from jax.experimental.pallas import tpu as pltpu
from jax.experimental.pallas import tpu_sc as plsc

NN = 10000
NP = 10240
HH = 5
NB = 64
EBLK = 128
NCC = 2
NSS = 16
NWK = NCC * NSS
RPT = NP // NSS
CH = 80

_F32 = jnp.float32
_HI = lax.Precision.HIGHEST


def _rnd(a):
    return a.astype(jnp.bfloat16).astype(_F32)


_GDN = lax.GatherDimensionNumbers(offset_dims=(), collapsed_slice_dims=(0,),
                                  start_index_map=(0,))


def _bcast(v16, r):
    idx = jnp.full((16, 1), r, jnp.int32)
    return lax.gather(v16, idx, _GDN, (1,),
                      mode=lax.GatherScatterMode.PROMISE_IN_BOUNDS)


def _zero_ref(ref, nwords):
    def body(i, _):
        ref[pl.ds(i * 16, 16)] = jnp.zeros((16,), _F32)
        return 0
    lax.fori_loop(0, nwords // 16, body, 0)


def _zero_2d(ref, rows, cols):
    def body(i, _):
        for c in range(cols // 16):
            ref[i, pl.ds(c * 16, 16)] = jnp.zeros((16,), _F32)
        return 0
    lax.fori_loop(0, rows, body, 0)


def _add_const(ref, nwords, val):
    def body(i, _):
        sl = pl.ds(i * 16, 16)
        ref[sl] = ref[sl] + val
        return 0
    lax.fori_loop(0, nwords // 16, body, 0)


def _build_deg(etp):
    epw = etp // NWK
    nblk = epw // EBLK
    mesh = plsc.VectorSubcoreMesh(core_axis_name="c", subcore_axis_name="s")

    @functools.partial(
        pl.kernel, mesh=mesh,
        compiler_params=pltpu.CompilerParams(needs_layout_passes=False),
        out_type=jax.ShapeDtypeStruct((NCC * NP,), _F32),
        scratch_types=[
            pltpu.VMEM((EBLK,), jnp.int32),
            pltpu.VMEM((EBLK,), _F32),
            pltpu.VMEM((RPT,), _F32),
            pltpu.VMEM_SHARED((NP,), _F32),
        ],
    )
    def deg_kernel(colp_hbm, out_hbm, cidx, onesv, stg, acc_sh):
        cid = lax.axis_index("c")
        sid = lax.axis_index("s")
        base = (cid * NSS + sid) * epw
        rbase = sid * RPT
        _zero_ref(stg, RPT)
        pltpu.sync_copy(stg, acc_sh.at[pl.ds(rbase, RPT)])
        _zero_ref(onesv, EBLK)
        _add_const(onesv, EBLK, 1.0)
        plsc.subcore_barrier()

        def blk(b, _):
            pltpu.sync_copy(colp_hbm.at[pl.ds(base + b * EBLK, EBLK)], cidx)
            pltpu.sync_copy(onesv, acc_sh.at[cidx], add=True)
            return 0
        lax.fori_loop(0, nblk, blk, 0)

        plsc.subcore_barrier()
        pltpu.sync_copy(acc_sh.at[pl.ds(rbase, RPT)], stg)
        pltpu.sync_copy(stg, out_hbm.at[pl.ds(cid * NP + rbase, RPT)])

    return deg_kernel


def _build_den(etp):
    epw = etp // NWK
    nblk = epw // EBLK
    mesh = plsc.VectorSubcoreMesh(core_axis_name="c", subcore_axis_name="s")

    @functools.partial(
        pl.kernel, mesh=mesh,
        compiler_params=pltpu.CompilerParams(needs_layout_passes=False),
        out_type=jax.ShapeDtypeStruct((NCC * HH * NP,), _F32),
        scratch_types=[
            pltpu.VMEM((NP,), _F32),
            pltpu.VMEM((NP,), _F32),
            pltpu.VMEM((EBLK,), jnp.int32),
            pltpu.VMEM((EBLK,), jnp.int32),
            pltpu.VMEM((EBLK,), _F32),
            pltpu.VMEM((RPT,), _F32),
            pltpu.VMEM_SHARED((NP,), _F32),
        ],
    )
    def den_kernel(ast_hbm, adt_hbm, rowp_hbm, colp_hbm, out_hbm,
                   asv, adv, ridx, cidx, exv, stg, acc_sh):
        cid = lax.axis_index("c")
        sid = lax.axis_index("s")
        base = (cid * NSS + sid) * epw
        rbase = sid * RPT

        for k in range(HH):
            pltpu.sync_copy(ast_hbm.at[pl.ds(k * NP, NP)], asv)
            pltpu.sync_copy(adt_hbm.at[pl.ds(k * NP, NP)], adv)
            _zero_ref(stg, RPT)
            pltpu.sync_copy(stg, acc_sh.at[pl.ds(rbase, RPT)])
            plsc.subcore_barrier()

            def blk(b, _):
                pltpu.sync_copy(rowp_hbm.at[pl.ds(base + b * EBLK, EBLK)], ridx)
                pltpu.sync_copy(colp_hbm.at[pl.ds(base + b * EBLK, EBLK)], cidx)
                def grp(j, _):
                    gsl = pl.ds(j * 16, 16)
                    a16 = (plsc.load_gather(asv, [ridx[gsl]]) +
                           plsc.load_gather(adv, [cidx[gsl]]))
                    a16 = jnp.where(a16 > 0, a16, 0.2 * a16)
                    exv[gsl] = jnp.exp(a16)
                    return 0
                lax.fori_loop(0, EBLK // 16, grp, 0)
                pltpu.sync_copy(exv, acc_sh.at[cidx], add=True)
                return 0
            lax.fori_loop(0, nblk, blk, 0)

            plsc.subcore_barrier()
            pltpu.sync_copy(acc_sh.at[pl.ds(rbase, RPT)], stg)
            pltpu.sync_copy(
                stg, out_hbm.at[pl.ds((cid * HH + k) * NP + rbase, RPT)])
            plsc.subcore_barrier()

    return den_kernel


def _build_wts_gcn(etp):
    eblk = 384
    epw = etp // NWK
    nblk = epw // eblk
    mesh = plsc.VectorSubcoreMesh(core_axis_name="c", subcore_axis_name="s")

    @functools.partial(
        pl.kernel, mesh=mesh,
        compiler_params=pltpu.CompilerParams(needs_layout_passes=False),
        out_type=jax.ShapeDtypeStruct((etp,), _F32),
        scratch_types=[
            pltpu.VMEM((NP,), _F32),
            pltpu.VMEM((384,), jnp.int32),
            pltpu.VMEM((384,), jnp.int32),
            pltpu.VMEM((384,), _F32),
        ],
    )
    def wts_kernel(dis_hbm, rowp_hbm, colp_hbm, out_hbm, disv, ridx, cidx, wb):
        cid = lax.axis_index("c")
        sid = lax.axis_index("s")
        base = (cid * NSS + sid) * epw
        pltpu.sync_copy(dis_hbm, disv)

        def blk(b, _):
            eb = base + b * eblk
            pltpu.sync_copy(rowp_hbm.at[pl.ds(eb, eblk)], ridx)
            pltpu.sync_copy(colp_hbm.at[pl.ds(eb, eblk)], cidx)
            def grp(j, _):
                gsl = pl.ds(j * 16, 16)
                wb[gsl] = (plsc.load_gather(disv, [ridx[gsl]]) *
                           plsc.load_gather(disv, [cidx[gsl]]))
                return 0
            lax.fori_loop(0, eblk // 16, grp, 0)
            pltpu.sync_copy(wb, out_hbm.at[pl.ds(eb, eblk)])
            return 0
        lax.fori_loop(0, nblk, blk, 0)

    return wts_kernel


def _build_wts_gat(etp):
    eblk = 384
    epw = etp // NWK
    nblk = epw // eblk
    mesh = plsc.VectorSubcoreMesh(core_axis_name="c", subcore_axis_name="s")

    @functools.partial(
        pl.kernel, mesh=mesh,
        compiler_params=pltpu.CompilerParams(needs_layout_passes=False),
        out_type=jax.ShapeDtypeStruct((HH * etp,), _F32),
        scratch_types=[
            pltpu.VMEM((NP,), _F32),
            pltpu.VMEM((NP,), _F32),
            pltpu.VMEM((NP,), _F32),
            pltpu.VMEM((384,), jnp.int32),
            pltpu.VMEM((384,), jnp.int32),
            pltpu.VMEM((384,), _F32),
        ],
    )
    def wts_kernel(ast_hbm, adt_hbm, den_hbm, rowp_hbm, colp_hbm, out_hbm,
                   asv, adv, denv, ridx, cidx, wb):
        cid = lax.axis_index("c")
        sid = lax.axis_index("s")
        base = (cid * NSS + sid) * epw

        for k in range(HH):
            pltpu.sync_copy(ast_hbm.at[pl.ds(k * NP, NP)], asv)
            pltpu.sync_copy(adt_hbm.at[pl.ds(k * NP, NP)], adv)
            pltpu.sync_copy(den_hbm.at[pl.ds(k * NP, NP)], denv)

            def blk(b, _):
                eb = base + b * eblk
                pltpu.sync_copy(rowp_hbm.at[pl.ds(eb, eblk)], ridx)
                pltpu.sync_copy(colp_hbm.at[pl.ds(eb, eblk)], cidx)
                def grp(j, _):
                    gsl = pl.ds(j * 16, 16)
                    r16 = ridx[gsl]
                    c16 = cidx[gsl]
                    a16 = (plsc.load_gather(asv, [r16]) +
                           plsc.load_gather(adv, [c16]))
                    a16 = jnp.where(a16 > 0, a16, 0.2 * a16)
                    ex = jnp.exp(a16)
                    den16 = plsc.load_gather(denv, [c16])
                    wb[gsl] = ex / (den16 + 1e-16)
                    return 0
                lax.fori_loop(0, eblk // 16, grp, 0)
                pltpu.sync_copy(wb, out_hbm.at[pl.ds(k * etp + eb, eblk)])
                return 0
            lax.fori_loop(0, nblk, blk, 0)

    return wts_kernel


def _build_prop(etp, mode):
    d = 128
    epw = etp // NWK
    nblk = epw // EBLK
    assert nblk % 2 == 0 and nblk >= 2
    mesh = plsc.VectorSubcoreMesh(core_axis_name="c", subcore_axis_name="s")

    if mode == "gcn":
        out_type = jax.ShapeDtypeStruct((NCC, NP, d), _F32)
        wshape = (etp,)
    else:
        out_type = jax.ShapeDtypeStruct((HH, NCC, NP, d), _F32)
        wshape = (HH * etp,)

    @functools.partial(
        pl.kernel, mesh=mesh,
        compiler_params=pltpu.CompilerParams(needs_layout_passes=False),
        out_type=out_type,
        scratch_types=[
            pltpu.VMEM((EBLK, d), _F32),
            pltpu.VMEM((EBLK, d), _F32),
            pltpu.VMEM((CH, d), _F32),
            pltpu.VMEM((EBLK,), jnp.int32),
            pltpu.VMEM((EBLK,), jnp.int32),
            pltpu.VMEM((EBLK,), jnp.int32),
            pltpu.VMEM((EBLK,), jnp.int32),
            pltpu.VMEM((EBLK,), _F32),
            pltpu.VMEM((EBLK,), _F32),
            pltpu.VMEM_SHARED((NP, d), _F32),
            pltpu.SemaphoreType.DMA,
            pltpu.SemaphoreType.DMA,
        ],
    )
    def prop_kernel(tbl_hbm, w_hbm, rowp_hbm, colp_hbm, out_hbm,
                    rows_a, rows_b, stg, ridx_a, ridx_b, cidx_a, cidx_b,
                    wv_a, wv_b, acc_sh, sem_a, sem_b):
        cid = lax.axis_index("c")
        sid = lax.axis_index("s")
        base = (cid * NSS + sid) * epw
        rbase = sid * RPT
        bufs = ((rows_a, ridx_a, cidx_a, wv_a, sem_a),
                (rows_b, ridx_b, cidx_b, wv_b, sem_b))

        def one_pass(woff, out_at):
            _zero_2d(stg, CH, d)
            for i in range(RPT // CH):
                pltpu.sync_copy(stg, acc_sh.at[pl.ds(rbase + i * CH, CH)])
            plsc.subcore_barrier()

            def issue(b, t):
                eb = base + b * EBLK
                pltpu.sync_copy(rowp_hbm.at[pl.ds(eb, EBLK)], t[1])
                pltpu.sync_copy(colp_hbm.at[pl.ds(eb, EBLK)], t[2])
                pltpu.sync_copy(w_hbm.at[pl.ds(woff + eb, EBLK)], t[3])
                pltpu.make_async_copy(tbl_hbm.at[t[1]], t[0], t[4]).start()

            def wait(t):
                pltpu.make_async_copy(tbl_hbm.at[t[1]], t[0], t[4]).wait()

            def crunch(t):
                def mgrp(j, _):
                    w16 = t[3][pl.ds(j * 16, 16)]
                    for r in range(16):
                        wb = _bcast(w16, r)
                        ri = j * 16 + r
                        for c in range(d // 16):
                            csl = pl.ds(c * 16, 16)
                            t[0][ri, csl] = t[0][ri, csl] * wb
                    return 0
                lax.fori_loop(0, EBLK // 16, mgrp, 0)
                pltpu.sync_copy(t[0], acc_sh.at[t[2]], add=True)

            issue(0, bufs[0])

            def body(i, _):
                b = 2 * i
                issue(b + 1, bufs[1])
                wait(bufs[0]); crunch(bufs[0])
                issue(b + 2, bufs[0])
                wait(bufs[1]); crunch(bufs[1])
                return 0
            lax.fori_loop(0, nblk // 2 - 1, body, 0)
            issue(nblk - 1, bufs[1])
            wait(bufs[0]); crunch(bufs[0])
            wait(bufs[1]); crunch(bufs[1])

            plsc.subcore_barrier()
            for i in range(RPT // CH):
                dsl = pl.ds(rbase + i * CH, CH)
                pltpu.sync_copy(acc_sh.at[dsl], stg)
                pltpu.sync_copy(stg, out_at(dsl))
            plsc.subcore_barrier()

        if mode == "gcn":
            one_pass(0, lambda dsl: out_hbm.at[cid, dsl])
        else:
            for k in range(HH):
                one_pass(k * etp, lambda dsl, k=k: out_hbm.at[k, cid, dsl])

    return prop_kernel


def _gelu(t):
    return 0.5 * t * (1.0 + lax.erf(t * (2.0 ** -0.5)))


def _bn_rows(h, g, b):
    mu = jnp.mean(h, axis=0)
    var = jnp.mean((h - mu) ** 2, axis=0)
    return (h - mu) / jnp.sqrt(var + 1e-5) * g + b


def _t_dis(degp):
    def body(degp_ref, dis_ref):
        deg = degp_ref[0, :] + degp_ref[1, :]
        dis_ref[...] = jnp.where(deg > 0, lax.rsqrt(deg), 0.0)
    return pl.pallas_call(
        body, out_shape=jax.ShapeDtypeStruct((NP,), _F32))(degp)


def _t0(x, w1):
    def body(x_ref, w_ref, o_ref):
        o_ref[...] = jnp.dot(x_ref[...], w_ref[...],
                             preferred_element_type=_F32)
    return pl.pallas_call(
        body, out_shape=jax.ShapeDtypeStruct((NN, 128), _F32))(x, w1)


def _t_den(denp):
    def body(dp_ref, d_ref):
        d_ref[...] = dp_ref[0, :] + dp_ref[1, :]
    return pl.pallas_call(
        body, out_shape=jax.ShapeDtypeStruct((HH * NP,), _F32))(denp)


def _t1(p1, b1, g1, be1):
    def body(p_ref, b_ref, g_ref, be_ref, h_ref):
        hh = p_ref[0, :NN, :] + p_ref[1, :NN, :] + b_ref[...]
        h_ref[...] = _gelu(_bn_rows(hh, g_ref[...], be_ref[...]))
    return pl.pallas_call(
        body, out_shape=jax.ShapeDtypeStruct((NN, 128), _F32))(
            p1, b1, g1, be1)


def _t2(p2, wh, bh, gh, beh, h1, wg3, ats, atd):
    def body(p_ref, w_ref, b_ref, g_ref, be_ref, h1_ref, wg_ref, ats_ref,
             atd_ref, h2_ref, ast_ref, adt_ref):
        xx = p_ref[0, :NN, :] + p_ref[1, :NN, :]
        hh = jnp.dot(xx, w_ref[...], preferred_element_type=_F32,
                     precision=_HI) + b_ref[...]
        h2 = _gelu(_bn_rows(hh, g_ref[...], be_ref[...])) + h1_ref[...]
        h2_ref[...] = h2
        h2r = h2.astype(jnp.bfloat16).astype(_F32)
        va_s = jnp.sum(wg_ref[...] * ats_ref[...][None, :, :], axis=-1)
        va_d = jnp.sum(wg_ref[...] * atd_ref[...][None, :, :], axis=-1)
        a_sT = jax.lax.dot_general(va_s, h2r, (((0,), (1,)), ((), ())),
                                   preferred_element_type=_F32,
                                   precision=_HI)
        a_dT = jax.lax.dot_general(va_d, h2r, (((0,), (1,)), ((), ())),
                                   preferred_element_type=_F32,
                                   precision=_HI)
        pad = jnp.zeros((HH, NP - NN), _F32)
        ast_ref[...] = jnp.concatenate([a_sT, pad], axis=1)
        adt_ref[...] = jnp.concatenate([a_dT, pad], axis=1)
    return pl.pallas_call(
        body,
        out_shape=[jax.ShapeDtypeStruct((NN, 128), _F32),
                   jax.ShapeDtypeStruct((HH, NP), _F32),
                   jax.ShapeDtypeStruct((HH, NP), _F32)])(
            p2, wh, bh, gh, beh, h1, wg3, ats, atd)


def _t4(pg, wgh, bg):
    def body(p_ref, w_ref, bg_ref, g_ref):
        k = pl.program_id(0)
        pk = p_ref[0, 0, :NN, :] + p_ref[0, 1, :NN, :]
        contrib = jnp.dot(pk, w_ref[0], preferred_element_type=_F32,
                          precision=_HI) * (1.0 / HH)
        @pl.when(k == 0)
        def _():
            g_ref[...] = contrib + bg_ref[...]
        @pl.when(k > 0)
        def _():
            g_ref[...] = g_ref[...] + contrib
    return pl.pallas_call(
        body,
        grid=(HH,),
        in_specs=[pl.BlockSpec((1, NCC, NP, 128), lambda k: (k, 0, 0, 0)),
                  pl.BlockSpec((1, 128, 128), lambda k: (k, 0, 0)),
                  pl.BlockSpec((128,), lambda k: (0,))],
        out_specs=pl.BlockSpec((NN, 128), lambda k: (0, 0)),
        out_shape=jax.ShapeDtypeStruct((NN, 128), _F32))(pg, wgh, bg)


def _t5(p4, wo, bo, go, beo, batchf):
    def body(p_ref, w_ref, b_ref, g_ref, be_ref, bt_ref, s_ref, st_ref):
        xx = p_ref[0, :NN, :] + p_ref[1, :NN, :]
        hh = jnp.dot(xx, w_ref[...], preferred_element_type=_F32,
                     precision=_HI) + b_ref[...]
        hj = _gelu(_bn_rows(hh, g_ref[...], be_ref[...]))
        oh = (bt_ref[...] == lax.broadcasted_iota(jnp.int32, (NN, NB), 1)
              .astype(_F32)).astype(_F32)
        s_ref[...] = jax.lax.dot_general(oh, hj, (((0,), (0,)), ((), ())),
                                         preferred_element_type=_F32,
                                         precision=_HI)
        lane = lax.broadcasted_iota(jnp.int32, (1, 1, 128), 2)
        stats = jnp.where(lane == 0, jnp.sum(hj),
                          jnp.where(lane == 1, jnp.sum(hj * hj), 0.0))
        st_ref[...] = stats
    return pl.pallas_call(
        body,
        grid=(4,),
        in_specs=[pl.BlockSpec((NCC, NP, 128), lambda j: (0, 0, 0)),
                  pl.BlockSpec((128, 128), lambda j: (0, j)),
                  pl.BlockSpec((128,), lambda j: (j,)),
                  pl.BlockSpec((128,), lambda j: (j,)),
                  pl.BlockSpec((128,), lambda j: (j,)),
                  pl.BlockSpec((NN, 1), lambda j: (0, 0))],
        out_specs=[pl.BlockSpec((NB, 128), lambda j: (0, j)),
                   pl.BlockSpec((1, 1, 128), lambda j: (j, 0, 0))],
        out_shape=[jax.ShapeDtypeStruct((NB, 512), _F32),
                   jax.ShapeDtypeStruct((4, 1, 128), _F32)])(
            p4, wo, bo, go, beo, batchf)


def _t6(s, stats, batchf, lnw, lnb, pw, pb, png, pnb, fw, fb, l2w, l2b, cw, cb):
    def body(s_ref, st_ref, bt_ref, lnw_ref, lnb_ref, pw_ref, pb_ref, png_ref,
             pnb_ref, fw_ref, fb_ref, l2w_ref, l2b_ref, cw_ref, cb_ref,
             xn_ref, c_ref):
        oh = (bt_ref[...] == lax.broadcasted_iota(jnp.int32, (NN, NB), 1)
              .astype(_F32)).astype(_F32)
        cnt = jnp.sum(oh, axis=0)[:, None]
        g = s_ref[...] / jnp.maximum(cnt, 1.0)
        tot = float(NN * 512)
        musum = jnp.sum(st_ref[:, 0, 0])
        sqsum = jnp.sum(st_ref[:, 0, 1])
        mu = musum / tot
        var = sqsum / tot - mu * mu
        g = (g - mu) / jnp.sqrt(var + 1e-5) * lnw_ref[...] + lnb_ref[...]
        p = jnp.dot(g, pw_ref[...], preferred_element_type=_F32) + pb_ref[...]
        p = _gelu(_bn_rows(p, png_ref[...], pnb_ref[...]))
        q = jnp.dot(p, fw_ref[...], preferred_element_type=_F32) + fb_ref[...] + p
        mu2 = jnp.mean(q, axis=-1, keepdims=True)
        var2 = jnp.mean((q - mu2) ** 2, axis=-1, keepdims=True)
        z = (q - mu2) / jnp.sqrt(var2 + 1e-5) * l2w_ref[...] + l2b_ref[...]
        nrm = jnp.sqrt(jnp.sum(z * z, axis=1, keepdims=True))
        xn = z / jnp.maximum(nrm, 1e-12)
        xn_ref[...] = xn
        lg = jnp.dot(xn, cw_ref[...], preferred_element_type=_F32) + cb_ref[...]
        m = jnp.max(lg, axis=1, keepdims=True)
        lse = m + jnp.log(jnp.sum(jnp.exp(lg - m), axis=1, keepdims=True))
        c_ref[...] = lg - lse
    return pl.pallas_call(
        body,
        out_shape=[jax.ShapeDtypeStruct((NB, 128), _F32),
                   jax.ShapeDtypeStruct((NB, 10), _F32)])(
            s, stats, batchf, lnw, lnb, pw, pb, png, pnb, fw, fb, l2w, l2b,
            cw, cb)


def kernel(x, W1, b1, g1, be1, Wh, bh, gh, beh, Wg, att_s, att_d, bg, Wo, bo,
           go, beo, lnw, lnb, pW, pb, png, pnb, fW, fb, l2w, l2b, cW, cb,
           edge_index, batch):
    n = x.shape[0]
    e = edge_index.shape[1]
    et = e + n
    ealign = NWK * EBLK
    etp = ((et + ealign - 1) // ealign) * ealign

    sl = jnp.arange(n, dtype=edge_index.dtype)
    rowp = jnp.pad(jnp.concatenate([edge_index[0], sl]), (0, etp - et))
    colp = jnp.pad(jnp.concatenate([edge_index[1], sl]), (0, etp - et),
                   constant_values=n)

    wg3_r = _rnd(Wg.reshape(128, HH, 128))
    wgh_r = jnp.transpose(wg3_r, (1, 0, 2))
    wh_r = _rnd(Wh)
    wo_r = _rnd(Wo)
    batchf = batch.astype(_F32)[:, None]

    degp = _build_deg(etp)(colp).reshape(NCC, NP)
    dis = _t_dis(degp)

    wgcn = _build_wts_gcn(etp)(dis, rowp, colp)

    x1 = _t0(x, W1)
    p1 = _build_prop(etp, "gcn")(x1, wgcn, rowp, colp)
    h1 = _t1(p1, b1, g1, be1)

    h1r = _rnd(h1)
    p2 = _build_prop(etp, "gcn")(h1r, wgcn, rowp, colp)
    h2, a_sT, a_dT = _t2(p2, wh_r, bh, gh, beh, h1, wg3_r, att_s, att_d)

    astf = a_sT.reshape(HH * NP)
    adtf = a_dT.reshape(HH * NP)
    denp = _build_den(etp)(astf, adtf, rowp, colp)
    den = _t_den(denp.reshape(NCC, HH * NP))
    wgat = _build_wts_gat(etp)(astf, adtf, den, rowp, colp)
    h2r = _rnd(h2)
    pg = _build_prop(etp, "gat")(h2r, wgat, rowp, colp)
    g = _t4(pg, wgh_r, bg)

    gr = _rnd(g)
    p4 = _build_prop(etp, "gcn")(gr, wgcn, rowp, colp)
    s, stats = _t5(p4, wo_r, bo, go, beo, batchf)
    xn, c = _t6(s, stats, batchf, lnw, lnb, pW, pb, png, pnb, fW, fb,
                l2w, l2b, cW, cb)
    return xn, c
